# baseline (speedup 1.0000x reference)
import functools

import jax
import jax.numpy as jnp
from jax import lax
from jax.experimental import pallas as pl
from jax.experimental.pallas import tpu as pltpu
from jax.experimental.pallas import tpu_sc as plsc

N = 10000
E = 320000
D = 128

NC = 2
NS = 16
NW = NC * NS
CH = 128
EPW = 10240
NCHUNK = EPW // CH
EPAD = NW * EPW + 2 * CH
NPAD = 10240
RPT = NPAD // NS

BN = 1000
GRID = N // BN

_mesh = plsc.VectorSubcoreMesh(core_axis_name="c", subcore_axis_name="s")


@functools.partial(
    pl.kernel,
    mesh=_mesh,
    out_type=jax.ShapeDtypeStruct((NC * 2 * N,), jnp.float32),
    scratch_types=[
        pltpu.VMEM_SHARED((2 * N + 8,), jnp.float32),
        pltpu.VMEM((2, CH), jnp.int32),
        pltpu.VMEM((2, CH), jnp.int32),
        pltpu.VMEM((CH,), jnp.float32),
        pltpu.VMEM((2000,), jnp.float32),
        pltpu.SemaphoreType.DMA,
        pltpu.SemaphoreType.DMA,
    ],
)
def _deg_kernel(src_hbm, dstoff_hbm, out_hbm,
                deg_sp, src_v, dst_v, ones_v, stage_v, s1sem, s2sem):
    c = lax.axis_index("c")
    s = lax.axis_index("s")
    wid = c * NS + s
    base = wid * EPW

    pltpu.sync_copy(src_hbm.at[pl.ds(base, CH)], src_v.at[0])
    pltpu.sync_copy(dstoff_hbm.at[pl.ds(base, CH)], dst_v.at[0])
    off1 = pl.multiple_of(base + CH, 8)
    pltpu.make_async_copy(src_hbm.at[pl.ds(off1, CH)], src_v.at[1], s1sem).start()
    pltpu.make_async_copy(dstoff_hbm.at[pl.ds(off1, CH)], dst_v.at[1], s2sem).start()

    @pl.when(s < 10)
    def _():
        def zb(i, carry):
            stage_v[pl.ds(i * 16, 16)] = jnp.zeros((16,), jnp.float32)
            return carry
        lax.fori_loop(0, 2000 // 16, zb, 0)
        off = pl.multiple_of(s * 2000, 8)
        pltpu.sync_copy(stage_v, deg_sp.at[pl.ds(off, 2000)])

    for j in range(CH // 16):
        ones_v[pl.ds(j * 16, 16)] = jnp.full((16,), 1.0, jnp.float32)

    plsc.subcore_barrier()

    def body(i, carry):
        b = lax.rem(i, 2)
        offp = pl.multiple_of(base + (i + 2) * CH, 8)
        pltpu.make_async_copy(
            src_hbm.at[pl.ds(offp, CH)], src_v.at[b], s1sem).wait()
        pltpu.make_async_copy(
            dstoff_hbm.at[pl.ds(offp, CH)], dst_v.at[b], s2sem).wait()
        pltpu.sync_copy(ones_v, deg_sp.at[src_v.at[b]], add=True)
        pltpu.sync_copy(ones_v, deg_sp.at[dst_v.at[b]], add=True)
        pltpu.make_async_copy(
            src_hbm.at[pl.ds(offp, CH)], src_v.at[b], s1sem).start()
        pltpu.make_async_copy(
            dstoff_hbm.at[pl.ds(offp, CH)], dst_v.at[b], s2sem).start()
        return carry

    lax.fori_loop(0, NCHUNK, body, 0)
    pltpu.make_async_copy(
        src_hbm.at[pl.ds(base, CH)], src_v.at[0], s1sem).wait()
    pltpu.make_async_copy(
        dstoff_hbm.at[pl.ds(base, CH)], dst_v.at[0], s2sem).wait()
    plsc.subcore_barrier()

    @pl.when(s < 10)
    def _():
        off = pl.multiple_of(s * 2000, 8)
        pltpu.sync_copy(deg_sp.at[pl.ds(off, 2000)], stage_v)
        pltpu.sync_copy(stage_v, out_hbm.at[pl.ds(c * 2 * N + off, 2000)])


@functools.partial(
    pl.kernel,
    mesh=_mesh,
    out_type=jax.ShapeDtypeStruct((NC, NPAD, D), jnp.float32),
    scratch_types=[
        pltpu.VMEM_SHARED((NPAD, D), jnp.float32),
        pltpu.VMEM((3, CH), jnp.int32),
        pltpu.VMEM((2, CH), jnp.int32),
        pltpu.VMEM((2, CH, D), jnp.float32),
        pltpu.SemaphoreType.DMA,
        pltpu.SemaphoreType.DMA,
        pltpu.SemaphoreType.DMA,
    ],
)
def _seg_kernel(xs_hbm, src_hbm, dst_hbm, out_hbm,
                agg_sp, src_v, dst_v, rows_v, gsem, s1sem, s2sem):
    c = lax.axis_index("c")
    s = lax.axis_index("s")
    wid = c * NS + s
    base = wid * EPW

    pltpu.sync_copy(src_hbm.at[pl.ds(base, CH)], src_v.at[0])
    pltpu.sync_copy(dst_hbm.at[pl.ds(base, CH)], dst_v.at[0])
    pltpu.make_async_copy(xs_hbm.at[src_v.at[0]], rows_v.at[0], gsem).start()
    off1 = pl.multiple_of(base + CH, 8)
    pltpu.make_async_copy(src_hbm.at[pl.ds(off1, CH)], src_v.at[1], s1sem).start()
    pltpu.make_async_copy(dst_hbm.at[pl.ds(off1, CH)], dst_v.at[1], s2sem).start()

    def zb(i, carry):
        for j in range(D // 16):
            rows_v[1, i, pl.ds(j * 16, 16)] = jnp.zeros((16,), jnp.float32)
        return carry
    lax.fori_loop(0, CH, zb, 0)
    roff = pl.multiple_of(s * RPT, 8)
    for k in range(RPT // CH):
        pltpu.sync_copy(rows_v.at[1], agg_sp.at[pl.ds(roff + k * CH, CH)])

    plsc.subcore_barrier()

    def body(i, carry):
        b2 = lax.rem(i, 2)
        b2n = lax.rem(i + 1, 2)
        b3 = lax.rem(i, 3)
        b3n = lax.rem(i + 1, 3)
        b3p = lax.rem(i + 2, 3)
        offn = pl.multiple_of(base + (i + 1) * CH, 8)
        offp = pl.multiple_of(base + (i + 2) * CH, 8)
        pltpu.make_async_copy(
            xs_hbm.at[src_v.at[b3]], rows_v.at[b2], gsem).wait()
        pltpu.make_async_copy(
            src_hbm.at[pl.ds(offn, CH)], src_v.at[b3n], s1sem).wait()
        pltpu.make_async_copy(
            xs_hbm.at[src_v.at[b3n]], rows_v.at[b2n], gsem).start()
        pltpu.make_async_copy(
            src_hbm.at[pl.ds(offp, CH)], src_v.at[b3p], s1sem).start()
        pltpu.make_async_copy(
            dst_hbm.at[pl.ds(offn, CH)], dst_v.at[b2n], s2sem).wait()
        pltpu.sync_copy(rows_v.at[b2], agg_sp.at[dst_v.at[b2]], add=True)
        pltpu.make_async_copy(
            dst_hbm.at[pl.ds(offp, CH)], dst_v.at[b2], s2sem).start()
        return carry

    lax.fori_loop(0, NCHUNK, body, 0)
    bl2 = NCHUNK % 2
    bl3 = NCHUNK % 3
    pltpu.make_async_copy(
        xs_hbm.at[src_v.at[bl3]], rows_v.at[bl2], gsem).wait()
    pltpu.make_async_copy(
        src_hbm.at[pl.ds(base, CH)], src_v.at[0], s1sem).wait()
    pltpu.make_async_copy(
        dst_hbm.at[pl.ds(base, CH)], dst_v.at[0], s2sem).wait()
    plsc.subcore_barrier()

    droff = pl.multiple_of(s * RPT, 8)
    for k in range(RPT // CH):
        pltpu.sync_copy(agg_sp.at[pl.ds(droff + k * CH, CH)], rows_v.at[0])
        pltpu.sync_copy(rows_v.at[0], out_hbm.at[c, pl.ds(droff + k * CH, CH)])


def _prep_body(deg_ref, x_ref, xs_ref, ns_ref, nd_ref):
    d = deg_ref[...]
    deg_o = d[0, 0] + d[1, 0]
    deg_i = d[0, 1] + d[1, 1]
    ns = lax.rsqrt(jnp.maximum(deg_o, 1.0))
    nd = lax.rsqrt(jnp.maximum(deg_i, 1.0))
    xs_ref[...] = x_ref[...] * ns
    ns_ref[...] = ns
    nd_ref[...] = nd


_prep_call = pl.pallas_call(
    _prep_body,
    grid=(GRID,),
    in_specs=[
        pl.BlockSpec((NC, 2, BN, 1), lambda i: (0, 0, i, 0)),
        pl.BlockSpec((BN, D), lambda i: (i, 0)),
    ],
    out_specs=[
        pl.BlockSpec((BN, D), lambda i: (i, 0)),
        pl.BlockSpec((BN, 1), lambda i: (i, 0)),
        pl.BlockSpec((BN, 1), lambda i: (i, 0)),
    ],
    out_shape=[
        jax.ShapeDtypeStruct((N, D), jnp.float32),
        jax.ShapeDtypeStruct((N, 1), jnp.float32),
        jax.ShapeDtypeStruct((N, 1), jnp.float32),
    ],
)


def _mid_body(p_ref, nd_ref, ns_ref, w_ref, b_ref, o_ref):
    agg = (p_ref[0] + p_ref[1]) * nd_ref[...]
    z = jnp.dot(agg, w_ref[...], preferred_element_type=jnp.float32)
    z = jnp.maximum(z + b_ref[...], 0.0)
    o_ref[...] = z * ns_ref[...]


_mid_call = pl.pallas_call(
    _mid_body,
    grid=(GRID,),
    in_specs=[
        pl.BlockSpec((NC, BN, D), lambda i: (0, i, 0)),
        pl.BlockSpec((BN, 1), lambda i: (i, 0)),
        pl.BlockSpec((BN, 1), lambda i: (i, 0)),
        pl.BlockSpec((D, D), lambda i: (0, 0)),
        pl.BlockSpec((1, D), lambda i: (0, 0)),
    ],
    out_specs=pl.BlockSpec((BN, D), lambda i: (i, 0)),
    out_shape=jax.ShapeDtypeStruct((N, D), jnp.float32),
)


def _fin_body(p_ref, nd_ref, w_ref, b_ref, o_ref):
    agg = (p_ref[0] + p_ref[1]) * nd_ref[...]
    z = jnp.dot(agg, w_ref[...], preferred_element_type=jnp.float32)
    o_ref[...] = jnp.maximum(z + b_ref[...], 0.0)


_fin_call = pl.pallas_call(
    _fin_body,
    grid=(GRID,),
    in_specs=[
        pl.BlockSpec((NC, BN, D), lambda i: (0, i, 0)),
        pl.BlockSpec((BN, 1), lambda i: (i, 0)),
        pl.BlockSpec((D, D), lambda i: (0, 0)),
        pl.BlockSpec((1, D), lambda i: (0, 0)),
    ],
    out_specs=pl.BlockSpec((BN, D), lambda i: (i, 0)),
    out_shape=jax.ShapeDtypeStruct((N, D), jnp.float32),
)


def kernel(x, edge_index, W1, b1, W2, b2):
    src = edge_index[0]
    dst = edge_index[1]
    npad = EPAD - E
    src_s = jnp.concatenate([src, jnp.zeros((npad,), jnp.int32)])
    dst_s = jnp.concatenate([dst, jnp.full((npad,), N, jnp.int32)])
    src_d = jnp.concatenate([src, jnp.full((npad,), 2 * N, jnp.int32)])
    dst_d = jnp.concatenate([dst + N, jnp.full((npad,), 2 * N, jnp.int32)])

    degs = _deg_kernel(src_d, dst_d)
    xs, ns, nd = _prep_call(degs.reshape(NC, 2, N, 1), x)

    p1 = _seg_kernel(xs, src_s, dst_s)
    zs = _mid_call(p1, nd, ns, W1, b1.reshape(1, D))

    p2 = _seg_kernel(zs, src_s, dst_s)
    out = _fin_call(p2, nd, W2, b2.reshape(1, D))
    return out


# speedup vs baseline: 2.8169x; 2.8169x over previous
import functools

import jax
import jax.numpy as jnp
from jax import lax
from jax.experimental import pallas as pl
from jax.experimental.pallas import tpu as pltpu
from jax.experimental.pallas import tpu_sc as plsc

N = 10000
E = 320000
D = 128

NC = 2
NS = 16
NW = NC * NS
CH = 128
EPW = 10240
NCHUNK = EPW // CH
EPAD = NW * EPW + 2 * CH
NPAD = 10240
RPT = NPAD // NS

BN = 1000
GRID = N // BN

_mesh = plsc.VectorSubcoreMesh(core_axis_name="c", subcore_axis_name="s")


@functools.partial(
    pl.kernel,
    mesh=_mesh,
    out_type=jax.ShapeDtypeStruct((NC * 2 * N,), jnp.float32),
    scratch_types=[
        pltpu.VMEM_SHARED((2 * N + 8,), jnp.float32),
        pltpu.VMEM((2, CH), jnp.int32),
        pltpu.VMEM((2, CH), jnp.int32),
        pltpu.VMEM((CH,), jnp.float32),
        pltpu.VMEM((2000,), jnp.float32),
        pltpu.SemaphoreType.DMA,
        pltpu.SemaphoreType.DMA,
    ],
)
def _deg_kernel(src_hbm, dstoff_hbm, out_hbm,
                deg_sp, src_v, dst_v, ones_v, stage_v, s1sem, s2sem):
    c = lax.axis_index("c")
    s = lax.axis_index("s")
    wid = c * NS + s
    base = wid * EPW

    pltpu.sync_copy(src_hbm.at[pl.ds(base, CH)], src_v.at[0])
    pltpu.sync_copy(dstoff_hbm.at[pl.ds(base, CH)], dst_v.at[0])
    off1 = pl.multiple_of(base + CH, 8)
    pltpu.make_async_copy(src_hbm.at[pl.ds(off1, CH)], src_v.at[1], s1sem).start()
    pltpu.make_async_copy(dstoff_hbm.at[pl.ds(off1, CH)], dst_v.at[1], s2sem).start()

    @pl.when(s < 10)
    def _():
        def zb(i, carry):
            stage_v[pl.ds(i * 16, 16)] = jnp.zeros((16,), jnp.float32)
            return carry
        lax.fori_loop(0, 2000 // 16, zb, 0)
        off = pl.multiple_of(s * 2000, 8)
        pltpu.sync_copy(stage_v, deg_sp.at[pl.ds(off, 2000)])

    for j in range(CH // 16):
        ones_v[pl.ds(j * 16, 16)] = jnp.full((16,), 1.0, jnp.float32)

    plsc.subcore_barrier()

    def body(i, carry):
        b = lax.rem(i, 2)
        offp = pl.multiple_of(base + (i + 2) * CH, 8)
        pltpu.make_async_copy(
            src_hbm.at[pl.ds(offp, CH)], src_v.at[b], s1sem).wait()
        pltpu.make_async_copy(
            dstoff_hbm.at[pl.ds(offp, CH)], dst_v.at[b], s2sem).wait()
        pltpu.sync_copy(ones_v, deg_sp.at[src_v.at[b]], add=True)
        pltpu.sync_copy(ones_v, deg_sp.at[dst_v.at[b]], add=True)
        pltpu.make_async_copy(
            src_hbm.at[pl.ds(offp, CH)], src_v.at[b], s1sem).start()
        pltpu.make_async_copy(
            dstoff_hbm.at[pl.ds(offp, CH)], dst_v.at[b], s2sem).start()
        return carry

    lax.fori_loop(0, NCHUNK, body, 0)
    pltpu.make_async_copy(
        src_hbm.at[pl.ds(base, CH)], src_v.at[0], s1sem).wait()
    pltpu.make_async_copy(
        dstoff_hbm.at[pl.ds(base, CH)], dst_v.at[0], s2sem).wait()
    plsc.subcore_barrier()

    @pl.when(s < 10)
    def _():
        off = pl.multiple_of(s * 2000, 8)
        pltpu.sync_copy(deg_sp.at[pl.ds(off, 2000)], stage_v)
        pltpu.sync_copy(stage_v, out_hbm.at[pl.ds(c * 2 * N + off, 2000)])


@functools.partial(
    pl.kernel,
    mesh=_mesh,
    out_type=jax.ShapeDtypeStruct((NC, NPAD, D), jnp.float32),
    scratch_types=[
        pltpu.VMEM_SHARED((NPAD, D), jnp.float32),
        pltpu.VMEM((3, CH), jnp.int32),
        pltpu.VMEM((2, CH), jnp.int32),
        pltpu.VMEM((2, CH, D), jnp.float32),
        pltpu.SemaphoreType.DMA,
        pltpu.SemaphoreType.DMA,
        pltpu.SemaphoreType.DMA,
    ],
)
def _seg_kernel(xs_hbm, src_hbm, dst_hbm, out_hbm,
                agg_sp, src_v, dst_v, rows_v, gsem, s1sem, s2sem):
    c = lax.axis_index("c")
    s = lax.axis_index("s")
    wid = c * NS + s
    base = wid * EPW

    pltpu.sync_copy(src_hbm.at[pl.ds(base, CH)], src_v.at[0])
    pltpu.sync_copy(dst_hbm.at[pl.ds(base, CH)], dst_v.at[0])
    pltpu.make_async_copy(xs_hbm.at[src_v.at[0]], rows_v.at[0], gsem).start()
    off1 = pl.multiple_of(base + CH, 8)
    pltpu.make_async_copy(src_hbm.at[pl.ds(off1, CH)], src_v.at[1], s1sem).start()
    pltpu.make_async_copy(dst_hbm.at[pl.ds(off1, CH)], dst_v.at[1], s2sem).start()

    def zb(i, carry):
        for j in range(D // 16):
            rows_v[1, i, pl.ds(j * 16, 16)] = jnp.zeros((16,), jnp.float32)
        return carry
    lax.fori_loop(0, CH, zb, 0)
    roff = pl.multiple_of(s * RPT, 8)
    for k in range(RPT // CH):
        pltpu.sync_copy(rows_v.at[1], agg_sp.at[pl.ds(roff + k * CH, CH)])

    plsc.subcore_barrier()

    def body(i, carry):
        b2 = lax.rem(i, 2)
        b2n = lax.rem(i + 1, 2)
        b3 = lax.rem(i, 3)
        b3n = lax.rem(i + 1, 3)
        b3p = lax.rem(i + 2, 3)
        offn = pl.multiple_of(base + (i + 1) * CH, 8)
        offp = pl.multiple_of(base + (i + 2) * CH, 8)
        pltpu.make_async_copy(
            xs_hbm.at[src_v.at[b3]], rows_v.at[b2], gsem).wait()
        pltpu.make_async_copy(
            src_hbm.at[pl.ds(offn, CH)], src_v.at[b3n], s1sem).wait()
        pltpu.make_async_copy(
            xs_hbm.at[src_v.at[b3n]], rows_v.at[b2n], gsem).start()
        pltpu.make_async_copy(
            src_hbm.at[pl.ds(offp, CH)], src_v.at[b3p], s1sem).start()
        pltpu.make_async_copy(
            dst_hbm.at[pl.ds(offn, CH)], dst_v.at[b2n], s2sem).wait()
        pltpu.sync_copy(rows_v.at[b2], agg_sp.at[dst_v.at[b2]], add=True)
        pltpu.make_async_copy(
            dst_hbm.at[pl.ds(offp, CH)], dst_v.at[b2], s2sem).start()
        return carry

    lax.fori_loop(0, NCHUNK, body, 0)
    bl2 = NCHUNK % 2
    bl3 = NCHUNK % 3
    pltpu.make_async_copy(
        xs_hbm.at[src_v.at[bl3]], rows_v.at[bl2], gsem).wait()
    pltpu.make_async_copy(
        src_hbm.at[pl.ds(base, CH)], src_v.at[0], s1sem).wait()
    pltpu.make_async_copy(
        dst_hbm.at[pl.ds(base, CH)], dst_v.at[0], s2sem).wait()
    plsc.subcore_barrier()

    droff = pl.multiple_of(s * RPT, 8)
    for k in range(RPT // CH):
        pltpu.sync_copy(agg_sp.at[pl.ds(droff + k * CH, CH)], rows_v.at[0])
        pltpu.sync_copy(rows_v.at[0], out_hbm.at[c, pl.ds(droff + k * CH, CH)])


def _prep_body(deg_ref, x_ref, xs_ref, ns_ref, nd_ref):
    d = deg_ref[...]
    deg_o = d[0, 0] + d[1, 0]
    deg_i = d[0, 1] + d[1, 1]
    ns = lax.rsqrt(jnp.maximum(deg_o, 1.0))
    nd = lax.rsqrt(jnp.maximum(deg_i, 1.0))
    xs_ref[...] = x_ref[...] * ns
    ns_ref[...] = ns
    nd_ref[...] = nd


_prep_call = pl.pallas_call(
    _prep_body,
    grid=(GRID,),
    in_specs=[
        pl.BlockSpec((NC, 2, BN, 1), lambda i: (0, 0, i, 0)),
        pl.BlockSpec((BN, D), lambda i: (i, 0)),
    ],
    out_specs=[
        pl.BlockSpec((BN, D), lambda i: (i, 0)),
        pl.BlockSpec((BN, 1), lambda i: (i, 0)),
        pl.BlockSpec((BN, 1), lambda i: (i, 0)),
    ],
    out_shape=[
        jax.ShapeDtypeStruct((N, D), jnp.float32),
        jax.ShapeDtypeStruct((N, 1), jnp.float32),
        jax.ShapeDtypeStruct((N, 1), jnp.float32),
    ],
)


def _mid_body(p_ref, nd_ref, ns_ref, w_ref, b_ref, o_ref):
    agg = (p_ref[0] + p_ref[1]) * nd_ref[...]
    z = jnp.dot(agg, w_ref[...], preferred_element_type=jnp.float32)
    z = jnp.maximum(z + b_ref[...], 0.0)
    o_ref[...] = z * ns_ref[...]


_mid_call = pl.pallas_call(
    _mid_body,
    grid=(GRID,),
    in_specs=[
        pl.BlockSpec((NC, BN, D), lambda i: (0, i, 0)),
        pl.BlockSpec((BN, 1), lambda i: (i, 0)),
        pl.BlockSpec((BN, 1), lambda i: (i, 0)),
        pl.BlockSpec((D, D), lambda i: (0, 0)),
        pl.BlockSpec((1, D), lambda i: (0, 0)),
    ],
    out_specs=pl.BlockSpec((BN, D), lambda i: (i, 0)),
    out_shape=jax.ShapeDtypeStruct((N, D), jnp.float32),
)


def _fin_body(p_ref, nd_ref, w_ref, b_ref, o_ref):
    agg = (p_ref[0] + p_ref[1]) * nd_ref[...]
    z = jnp.dot(agg, w_ref[...], preferred_element_type=jnp.float32)
    o_ref[...] = jnp.maximum(z + b_ref[...], 0.0)


_fin_call = pl.pallas_call(
    _fin_body,
    grid=(GRID,),
    in_specs=[
        pl.BlockSpec((NC, BN, D), lambda i: (0, i, 0)),
        pl.BlockSpec((BN, 1), lambda i: (i, 0)),
        pl.BlockSpec((D, D), lambda i: (0, 0)),
        pl.BlockSpec((1, D), lambda i: (0, 0)),
    ],
    out_specs=pl.BlockSpec((BN, D), lambda i: (i, 0)),
    out_shape=jax.ShapeDtypeStruct((N, D), jnp.float32),
)


def kernel(x, edge_index, W1, b1, W2, b2):
    src = edge_index[0]
    dst = edge_index[1]
    npad = EPAD - E
    iota = jnp.arange(npad, dtype=jnp.int32)
    src_s = jnp.concatenate([src, iota % 128])
    dst_s = jnp.concatenate([dst, N + iota % (NPAD - N)])
    src_d = jnp.concatenate([src, jnp.full((npad,), 2 * N, jnp.int32)])
    dst_d = jnp.concatenate([dst + N, jnp.full((npad,), 2 * N, jnp.int32)])

    degs = _deg_kernel(src_d, dst_d)
    xs, ns, nd = _prep_call(degs.reshape(NC, 2, N, 1), x)

    p1 = _seg_kernel(xs, src_s, dst_s)
    zs = _mid_call(p1, nd, ns, W1, b1.reshape(1, D))

    p2 = _seg_kernel(zs, src_s, dst_s)
    out = _fin_call(p2, nd, W2, b2.reshape(1, D))
    return out


# speedup vs baseline: 2.9273x; 1.0392x over previous
import functools

import jax
import jax.numpy as jnp
from jax import lax
from jax.experimental import pallas as pl
from jax.experimental.pallas import tpu as pltpu
from jax.experimental.pallas import tpu_sc as plsc

N = 10000
E = 320000
D = 128

NC = 2
NS = 16
NW = NC * NS
CH = 128
EPW = 10240
NCHUNK = EPW // CH
EPAD = NW * EPW + 2 * CH
NPAD = 10240
RPT = NPAD // NS

BN = 1000
GRID = N // BN

_mesh = plsc.VectorSubcoreMesh(core_axis_name="c", subcore_axis_name="s")


@functools.partial(
    pl.kernel,
    mesh=_mesh,
    out_type=jax.ShapeDtypeStruct((NC * 2 * N,), jnp.float32),
    scratch_types=[
        pltpu.VMEM_SHARED((2 * N + 2048,), jnp.float32),
        pltpu.VMEM((3, CH), jnp.int32),
        pltpu.VMEM((3, CH), jnp.int32),
        pltpu.VMEM((CH,), jnp.float32),
        pltpu.VMEM((2000,), jnp.float32),
        pltpu.SemaphoreType.DMA,
        pltpu.SemaphoreType.DMA,
        pltpu.SemaphoreType.DMA,
    ],
)
def _deg_kernel(src_hbm, dstoff_hbm, out_hbm,
                deg_sp, src_v, dst_v, ones_v, stage_v, s1sem, s2sem, ssem):
    c = lax.axis_index("c")
    s = lax.axis_index("s")
    wid = c * NS + s
    base = wid * EPW

    pltpu.sync_copy(src_hbm.at[pl.ds(base, CH)], src_v.at[0])
    pltpu.sync_copy(dstoff_hbm.at[pl.ds(base, CH)], dst_v.at[0])
    off1 = pl.multiple_of(base + CH, 8)
    pltpu.make_async_copy(src_hbm.at[pl.ds(off1, CH)], src_v.at[1], s1sem).start()
    pltpu.make_async_copy(dstoff_hbm.at[pl.ds(off1, CH)], dst_v.at[1], s2sem).start()

    @pl.when(s < 10)
    def _():
        def zb(i, carry):
            stage_v[pl.ds(i * 16, 16)] = jnp.zeros((16,), jnp.float32)
            return carry
        lax.fori_loop(0, 2000 // 16, zb, 0)
        off = pl.multiple_of(s * 2000, 8)
        pltpu.sync_copy(stage_v, deg_sp.at[pl.ds(off, 2000)])

    for j in range(CH // 16):
        ones_v[pl.ds(j * 16, 16)] = jnp.full((16,), 1.0, jnp.float32)

    plsc.subcore_barrier()

    def body(i, carry):
        b3 = lax.rem(i, 3)
        b3n = lax.rem(i + 1, 3)
        b3p = lax.rem(i + 2, 3)
        offp = pl.multiple_of(base + (i + 2) * CH, 8)

        @pl.when(i > 0)
        def _():
            pltpu.make_async_copy(
                ones_v, deg_sp.at[src_v.at[b3p]], ssem).wait()
            pltpu.make_async_copy(
                ones_v, deg_sp.at[dst_v.at[b3p]], ssem).wait()

        pltpu.make_async_copy(
            src_hbm.at[pl.ds(offp, CH)], src_v.at[b3], s1sem).wait()
        pltpu.make_async_copy(
            dstoff_hbm.at[pl.ds(offp, CH)], dst_v.at[b3], s2sem).wait()
        pltpu.make_async_copy(
            ones_v, deg_sp.at[src_v.at[b3]], ssem).start(add=True)
        pltpu.make_async_copy(
            ones_v, deg_sp.at[dst_v.at[b3]], ssem).start(add=True)
        pltpu.make_async_copy(
            src_hbm.at[pl.ds(offp, CH)], src_v.at[b3p], s1sem).start()
        pltpu.make_async_copy(
            dstoff_hbm.at[pl.ds(offp, CH)], dst_v.at[b3p], s2sem).start()
        return carry

    lax.fori_loop(0, NCHUNK, body, 0)
    pltpu.make_async_copy(
        ones_v, deg_sp.at[src_v.at[(NCHUNK - 1) % 3]], ssem).wait()
    pltpu.make_async_copy(
        ones_v, deg_sp.at[dst_v.at[(NCHUNK - 1) % 3]], ssem).wait()
    pltpu.make_async_copy(
        src_hbm.at[pl.ds(base, CH)], src_v.at[0], s1sem).wait()
    pltpu.make_async_copy(
        dstoff_hbm.at[pl.ds(base, CH)], dst_v.at[0], s2sem).wait()
    plsc.subcore_barrier()

    @pl.when(s < 10)
    def _():
        off = pl.multiple_of(s * 2000, 8)
        pltpu.sync_copy(deg_sp.at[pl.ds(off, 2000)], stage_v)
        pltpu.sync_copy(stage_v, out_hbm.at[pl.ds(c * 2 * N + off, 2000)])


@functools.partial(
    pl.kernel,
    mesh=_mesh,
    out_type=jax.ShapeDtypeStruct((NC, NPAD, D), jnp.float32),
    scratch_types=[
        pltpu.VMEM_SHARED((NPAD, D), jnp.float32),
        pltpu.VMEM((3, CH), jnp.int32),
        pltpu.VMEM((3, CH), jnp.int32),
        pltpu.VMEM((2, CH, D), jnp.float32),
        pltpu.SemaphoreType.DMA,
        pltpu.SemaphoreType.DMA,
        pltpu.SemaphoreType.DMA,
        pltpu.SemaphoreType.DMA,
    ],
)
def _seg_kernel(xs_hbm, src_hbm, dst_hbm, out_hbm,
                agg_sp, src_v, dst_v, rows_v, gsem, s1sem, s2sem, ssem):
    c = lax.axis_index("c")
    s = lax.axis_index("s")
    wid = c * NS + s
    base = wid * EPW

    pltpu.sync_copy(src_hbm.at[pl.ds(base, CH)], src_v.at[0])
    pltpu.sync_copy(dst_hbm.at[pl.ds(base, CH)], dst_v.at[0])
    pltpu.make_async_copy(xs_hbm.at[src_v.at[0]], rows_v.at[0], gsem).start()
    off1 = pl.multiple_of(base + CH, 8)
    pltpu.make_async_copy(src_hbm.at[pl.ds(off1, CH)], src_v.at[1], s1sem).start()
    pltpu.make_async_copy(dst_hbm.at[pl.ds(off1, CH)], dst_v.at[1], s2sem).start()

    def zb(i, carry):
        for j in range(D // 16):
            rows_v[1, i, pl.ds(j * 16, 16)] = jnp.zeros((16,), jnp.float32)
        return carry
    lax.fori_loop(0, CH, zb, 0)
    roff = pl.multiple_of(s * RPT, 8)
    for k in range(RPT // CH):
        pltpu.sync_copy(rows_v.at[1], agg_sp.at[pl.ds(roff + k * CH, CH)])

    plsc.subcore_barrier()

    def body(i, carry):
        b2 = lax.rem(i, 2)
        b2n = lax.rem(i + 1, 2)
        b3 = lax.rem(i, 3)
        b3n = lax.rem(i + 1, 3)
        b3p = lax.rem(i + 2, 3)
        offn = pl.multiple_of(base + (i + 1) * CH, 8)
        offp = pl.multiple_of(base + (i + 2) * CH, 8)
        pltpu.make_async_copy(
            xs_hbm.at[src_v.at[b3]], rows_v.at[b2], gsem).wait()

        @pl.when(i > 0)
        def _():
            pltpu.make_async_copy(
                rows_v.at[b2n], agg_sp.at[dst_v.at[b3n]], ssem).wait()

        pltpu.make_async_copy(
            src_hbm.at[pl.ds(offn, CH)], src_v.at[b3n], s1sem).wait()
        pltpu.make_async_copy(
            xs_hbm.at[src_v.at[b3n]], rows_v.at[b2n], gsem).start()
        pltpu.make_async_copy(
            src_hbm.at[pl.ds(offp, CH)], src_v.at[b3p], s1sem).start()
        pltpu.make_async_copy(
            dst_hbm.at[pl.ds(offn, CH)], dst_v.at[b3n], s2sem).wait()
        pltpu.make_async_copy(
            rows_v.at[b2], agg_sp.at[dst_v.at[b3]], ssem).start(add=True)
        pltpu.make_async_copy(
            dst_hbm.at[pl.ds(offp, CH)], dst_v.at[b3p], s2sem).start()
        return carry

    lax.fori_loop(0, NCHUNK, body, 0)
    bl2 = NCHUNK % 2
    bl3 = NCHUNK % 3
    pltpu.make_async_copy(
        xs_hbm.at[src_v.at[bl3]], rows_v.at[bl2], gsem).wait()
    pltpu.make_async_copy(
        rows_v.at[(NCHUNK - 1) % 2],
        agg_sp.at[dst_v.at[(NCHUNK - 1) % 3]], ssem).wait()
    pltpu.make_async_copy(
        src_hbm.at[pl.ds(base, CH)], src_v.at[0], s1sem).wait()
    pltpu.make_async_copy(
        dst_hbm.at[pl.ds(base, CH)], dst_v.at[0], s2sem).wait()
    plsc.subcore_barrier()

    droff = pl.multiple_of(s * RPT, 8)
    for k in range(RPT // CH):
        pltpu.sync_copy(agg_sp.at[pl.ds(droff + k * CH, CH)], rows_v.at[0])
        pltpu.sync_copy(rows_v.at[0], out_hbm.at[c, pl.ds(droff + k * CH, CH)])


def _prep_body(deg_ref, x_ref, xs_ref, ns_ref, nd_ref):
    d = deg_ref[...]
    deg_o = d[0, 0] + d[1, 0]
    deg_i = d[0, 1] + d[1, 1]
    ns = lax.rsqrt(jnp.maximum(deg_o, 1.0))
    nd = lax.rsqrt(jnp.maximum(deg_i, 1.0))
    xs_ref[...] = x_ref[...] * ns
    ns_ref[...] = ns
    nd_ref[...] = nd


_prep_call = pl.pallas_call(
    _prep_body,
    grid=(GRID,),
    in_specs=[
        pl.BlockSpec((NC, 2, BN, 1), lambda i: (0, 0, i, 0)),
        pl.BlockSpec((BN, D), lambda i: (i, 0)),
    ],
    out_specs=[
        pl.BlockSpec((BN, D), lambda i: (i, 0)),
        pl.BlockSpec((BN, 1), lambda i: (i, 0)),
        pl.BlockSpec((BN, 1), lambda i: (i, 0)),
    ],
    out_shape=[
        jax.ShapeDtypeStruct((N, D), jnp.float32),
        jax.ShapeDtypeStruct((N, 1), jnp.float32),
        jax.ShapeDtypeStruct((N, 1), jnp.float32),
    ],
)


def _mid_body(p_ref, nd_ref, ns_ref, w_ref, b_ref, o_ref):
    agg = (p_ref[0] + p_ref[1]) * nd_ref[...]
    z = jnp.dot(agg, w_ref[...], preferred_element_type=jnp.float32)
    z = jnp.maximum(z + b_ref[...], 0.0)
    o_ref[...] = z * ns_ref[...]


_mid_call = pl.pallas_call(
    _mid_body,
    grid=(GRID,),
    in_specs=[
        pl.BlockSpec((NC, BN, D), lambda i: (0, i, 0)),
        pl.BlockSpec((BN, 1), lambda i: (i, 0)),
        pl.BlockSpec((BN, 1), lambda i: (i, 0)),
        pl.BlockSpec((D, D), lambda i: (0, 0)),
        pl.BlockSpec((1, D), lambda i: (0, 0)),
    ],
    out_specs=pl.BlockSpec((BN, D), lambda i: (i, 0)),
    out_shape=jax.ShapeDtypeStruct((N, D), jnp.float32),
)


def _fin_body(p_ref, nd_ref, w_ref, b_ref, o_ref):
    agg = (p_ref[0] + p_ref[1]) * nd_ref[...]
    z = jnp.dot(agg, w_ref[...], preferred_element_type=jnp.float32)
    o_ref[...] = jnp.maximum(z + b_ref[...], 0.0)


_fin_call = pl.pallas_call(
    _fin_body,
    grid=(GRID,),
    in_specs=[
        pl.BlockSpec((NC, BN, D), lambda i: (0, i, 0)),
        pl.BlockSpec((BN, 1), lambda i: (i, 0)),
        pl.BlockSpec((D, D), lambda i: (0, 0)),
        pl.BlockSpec((1, D), lambda i: (0, 0)),
    ],
    out_specs=pl.BlockSpec((BN, D), lambda i: (i, 0)),
    out_shape=jax.ShapeDtypeStruct((N, D), jnp.float32),
)


def kernel(x, edge_index, W1, b1, W2, b2):
    src = edge_index[0]
    dst = edge_index[1]
    npad = EPAD - E
    iota = jnp.arange(npad, dtype=jnp.int32)
    src_s = jnp.concatenate([src, iota % 128])
    dst_s = jnp.concatenate([dst, N + iota % (NPAD - N)])
    src_d = jnp.concatenate([src, 2 * N + iota % 2048])
    dst_d = jnp.concatenate([dst + N, 2 * N + iota % 2048])

    degs = _deg_kernel(src_d, dst_d)
    xs, ns, nd = _prep_call(degs.reshape(NC, 2, N, 1), x)

    p1 = _seg_kernel(xs, src_s, dst_s)
    zs = _mid_call(p1, nd, ns, W1, b1.reshape(1, D))

    p2 = _seg_kernel(zs, src_s, dst_s)
    out = _fin_call(p2, nd, W2, b2.reshape(1, D))
    return out


# speedup vs baseline: 3.4598x; 1.1819x over previous
import functools

import jax
import jax.numpy as jnp
from jax import lax
from jax.experimental import pallas as pl
from jax.experimental.pallas import tpu as pltpu
from jax.experimental.pallas import tpu_sc as plsc

N = 10000
E = 320000
D = 128

NC = 2
NS = 16
NW = NC * NS
CH = 96
EPW = 10080
NCHUNK = EPW // CH
EPAD = NW * EPW + 4 * CH
NPAD = 10112
RPT = NPAD // NS

BN = 1000
GRID = N // BN

_mesh = plsc.VectorSubcoreMesh(core_axis_name="c", subcore_axis_name="s")


@functools.partial(
    pl.kernel,
    mesh=_mesh,
    out_type=jax.ShapeDtypeStruct((NC * 2 * N,), jnp.float32),
    scratch_types=[
        pltpu.VMEM_SHARED((2 * N + 2048,), jnp.float32),
        pltpu.VMEM((3, CH), jnp.int32),
        pltpu.VMEM((3, CH), jnp.int32),
        pltpu.VMEM((CH,), jnp.float32),
        pltpu.VMEM((2000,), jnp.float32),
        pltpu.SemaphoreType.DMA,
        pltpu.SemaphoreType.DMA,
        pltpu.SemaphoreType.DMA,
    ],
)
def _deg_kernel(src_hbm, dstoff_hbm, out_hbm,
                deg_sp, src_v, dst_v, ones_v, stage_v, s1sem, s2sem, ssem):
    c = lax.axis_index("c")
    s = lax.axis_index("s")
    wid = c * NS + s
    base = wid * EPW

    pltpu.sync_copy(src_hbm.at[pl.ds(base, CH)], src_v.at[0])
    pltpu.sync_copy(dstoff_hbm.at[pl.ds(base, CH)], dst_v.at[0])
    off1 = pl.multiple_of(base + CH, 8)
    pltpu.make_async_copy(src_hbm.at[pl.ds(off1, CH)], src_v.at[1], s1sem).start()
    pltpu.make_async_copy(dstoff_hbm.at[pl.ds(off1, CH)], dst_v.at[1], s2sem).start()

    @pl.when(s < 10)
    def _():
        def zb(i, carry):
            stage_v[pl.ds(i * 16, 16)] = jnp.zeros((16,), jnp.float32)
            return carry
        lax.fori_loop(0, 2000 // 16, zb, 0)
        off = pl.multiple_of(s * 2000, 8)
        pltpu.sync_copy(stage_v, deg_sp.at[pl.ds(off, 2000)])

    for j in range(CH // 16):
        ones_v[pl.ds(j * 16, 16)] = jnp.full((16,), 1.0, jnp.float32)

    plsc.subcore_barrier()

    def body(i, carry):
        b3 = lax.rem(i, 3)
        b3n = lax.rem(i + 1, 3)
        b3p = lax.rem(i + 2, 3)
        offp = pl.multiple_of(base + (i + 2) * CH, 8)

        @pl.when(i > 0)
        def _():
            pltpu.make_async_copy(
                ones_v, deg_sp.at[src_v.at[b3p]], ssem).wait()
            pltpu.make_async_copy(
                ones_v, deg_sp.at[dst_v.at[b3p]], ssem).wait()

        pltpu.make_async_copy(
            src_hbm.at[pl.ds(offp, CH)], src_v.at[b3], s1sem).wait()
        pltpu.make_async_copy(
            dstoff_hbm.at[pl.ds(offp, CH)], dst_v.at[b3], s2sem).wait()
        pltpu.make_async_copy(
            ones_v, deg_sp.at[src_v.at[b3]], ssem).start(add=True)
        pltpu.make_async_copy(
            ones_v, deg_sp.at[dst_v.at[b3]], ssem).start(add=True)
        pltpu.make_async_copy(
            src_hbm.at[pl.ds(offp, CH)], src_v.at[b3p], s1sem).start()
        pltpu.make_async_copy(
            dstoff_hbm.at[pl.ds(offp, CH)], dst_v.at[b3p], s2sem).start()
        return carry

    lax.fori_loop(0, NCHUNK, body, 0)
    pltpu.make_async_copy(
        ones_v, deg_sp.at[src_v.at[(NCHUNK - 1) % 3]], ssem).wait()
    pltpu.make_async_copy(
        ones_v, deg_sp.at[dst_v.at[(NCHUNK - 1) % 3]], ssem).wait()
    pltpu.make_async_copy(
        src_hbm.at[pl.ds(base, CH)], src_v.at[0], s1sem).wait()
    pltpu.make_async_copy(
        dstoff_hbm.at[pl.ds(base, CH)], dst_v.at[0], s2sem).wait()
    plsc.subcore_barrier()

    @pl.when(s < 10)
    def _():
        off = pl.multiple_of(s * 2000, 8)
        pltpu.sync_copy(deg_sp.at[pl.ds(off, 2000)], stage_v)
        pltpu.sync_copy(stage_v, out_hbm.at[pl.ds(c * 2 * N + off, 2000)])


@functools.partial(
    pl.kernel,
    mesh=_mesh,
    out_type=jax.ShapeDtypeStruct((NC, NPAD, D), jnp.float32),
    scratch_types=[
        pltpu.VMEM_SHARED((NPAD, D), jnp.float32),
        pltpu.VMEM((4, CH), jnp.int32),
        pltpu.VMEM((4, CH), jnp.int32),
        pltpu.VMEM((3, CH, D), jnp.float32),
        pltpu.SemaphoreType.DMA,
        pltpu.SemaphoreType.DMA,
        pltpu.SemaphoreType.DMA,
        pltpu.SemaphoreType.DMA,
        pltpu.SemaphoreType.DMA,
    ],
)
def _seg_kernel(xs_hbm, src_hbm, dst_hbm, out_hbm,
                agg_sp, src_v, dst_v, rows_v, ga, gb, s1sem, s2sem, ssem):
    c = lax.axis_index("c")
    s = lax.axis_index("s")
    wid = c * NS + s
    base = wid * EPW

    pltpu.sync_copy(src_hbm.at[pl.ds(base, CH)], src_v.at[0])
    pltpu.sync_copy(dst_hbm.at[pl.ds(base, CH)], dst_v.at[0])
    pltpu.make_async_copy(xs_hbm.at[src_v.at[0]], rows_v.at[0], ga).start()

    def _off(k):
        return pl.multiple_of(base + k * CH, 8)

    pltpu.make_async_copy(src_hbm.at[pl.ds(_off(1), CH)], src_v.at[1], s1sem).start()
    pltpu.make_async_copy(dst_hbm.at[pl.ds(_off(1), CH)], dst_v.at[1], s2sem).start()
    pltpu.make_async_copy(src_hbm.at[pl.ds(_off(1), CH)], src_v.at[1], s1sem).wait()
    pltpu.make_async_copy(xs_hbm.at[src_v.at[1]], rows_v.at[1], gb).start()
    pltpu.make_async_copy(src_hbm.at[pl.ds(_off(2), CH)], src_v.at[2], s1sem).start()
    pltpu.make_async_copy(src_hbm.at[pl.ds(_off(3), CH)], src_v.at[3], s1sem).start()

    def zb(i, carry):
        for j in range(D // 16):
            rows_v[2, i, pl.ds(j * 16, 16)] = jnp.zeros((16,), jnp.float32)
        return carry
    lax.fori_loop(0, CH, zb, 0)
    roff = pl.multiple_of(s * RPT, 8)
    for k in range(RPT // CH):
        pltpu.sync_copy(rows_v.at[2], agg_sp.at[pl.ds(roff + k * CH, CH)])
    pltpu.sync_copy(rows_v.at[2, pl.ds(0, RPT % CH)],
                    agg_sp.at[pl.ds(roff + (RPT // CH) * CH, RPT % CH)])

    plsc.subcore_barrier()

    def body(i, carry):
        b2 = lax.rem(i, 2)
        b3 = lax.rem(i, 3)
        b3p2 = lax.rem(i + 2, 3)
        b4 = lax.rem(i, 4)
        b4p2 = lax.rem(i + 2, 4)
        offp2 = pl.multiple_of(base + (i + 2) * CH, 8)
        offp4 = pl.multiple_of(base + (i + 4) * CH, 8)

        @pl.when(b2 == 0)
        def _():
            pltpu.make_async_copy(
                xs_hbm.at[src_v.at[b4]], rows_v.at[b3], ga).wait()

        @pl.when(b2 == 1)
        def _():
            pltpu.make_async_copy(
                xs_hbm.at[src_v.at[b4]], rows_v.at[b3], gb).wait()

        @pl.when(i > 0)
        def _():
            pltpu.make_async_copy(
                rows_v.at[b3p2], agg_sp.at[dst_v.at[b4p2]], ssem).wait()

        @pl.when(i == 0)
        def _():
            pltpu.make_async_copy(
                src_hbm.at[pl.ds(offp2, CH)], src_v.at[b4p2], s1sem).wait()

        pltpu.make_async_copy(
            src_hbm.at[pl.ds(offp2, CH)], src_v.at[b4p2], s1sem).wait()

        @pl.when(b2 == 0)
        def _():
            pltpu.make_async_copy(
                xs_hbm.at[src_v.at[b4p2]], rows_v.at[b3p2], ga).start()

        @pl.when(b2 == 1)
        def _():
            pltpu.make_async_copy(
                xs_hbm.at[src_v.at[b4p2]], rows_v.at[b3p2], gb).start()

        pltpu.make_async_copy(
            src_hbm.at[pl.ds(offp4, CH)], src_v.at[b4], s1sem).start()
        pltpu.make_async_copy(
            dst_hbm.at[pl.ds(offp2, CH)], dst_v.at[b4p2], s2sem).wait()
        pltpu.make_async_copy(
            rows_v.at[b3], agg_sp.at[dst_v.at[b4]], ssem).start(add=True)
        pltpu.make_async_copy(
            dst_hbm.at[pl.ds(offp2, CH)], dst_v.at[b4p2], s2sem).start()
        return carry

    lax.fori_loop(0, NCHUNK, body, 0)
    pltpu.make_async_copy(
        xs_hbm.at[src_v.at[NCHUNK % 4]], rows_v.at[NCHUNK % 3], ga).wait()
    pltpu.make_async_copy(
        xs_hbm.at[src_v.at[(NCHUNK + 1) % 4]],
        rows_v.at[(NCHUNK + 1) % 3], gb).wait()
    pltpu.make_async_copy(
        rows_v.at[(NCHUNK - 1) % 3],
        agg_sp.at[dst_v.at[(NCHUNK - 1) % 4]], ssem).wait()
    pltpu.make_async_copy(
        src_hbm.at[pl.ds(base, CH)], src_v.at[0], s1sem).wait()
    pltpu.make_async_copy(
        dst_hbm.at[pl.ds(base, CH)], dst_v.at[0], s2sem).wait()
    plsc.subcore_barrier()

    droff = pl.multiple_of(s * RPT, 8)
    for k in range(RPT // CH):
        pltpu.sync_copy(agg_sp.at[pl.ds(droff + k * CH, CH)], rows_v.at[0])
        pltpu.sync_copy(rows_v.at[0], out_hbm.at[c, pl.ds(droff + k * CH, CH)])
    pltpu.sync_copy(agg_sp.at[pl.ds(droff + (RPT // CH) * CH, RPT % CH)],
                    rows_v.at[0, pl.ds(0, RPT % CH)])
    pltpu.sync_copy(rows_v.at[0, pl.ds(0, RPT % CH)],
                    out_hbm.at[c, pl.ds(droff + (RPT // CH) * CH, RPT % CH)])


def _prep_body(deg_ref, x_ref, xs_ref, ns_ref, nd_ref):
    d = deg_ref[...]
    deg_o = d[0, 0] + d[1, 0]
    deg_i = d[0, 1] + d[1, 1]
    ns = lax.rsqrt(jnp.maximum(deg_o, 1.0))
    nd = lax.rsqrt(jnp.maximum(deg_i, 1.0))
    xs_ref[...] = x_ref[...] * ns
    ns_ref[...] = ns
    nd_ref[...] = nd


_prep_call = pl.pallas_call(
    _prep_body,
    grid=(GRID,),
    in_specs=[
        pl.BlockSpec((NC, 2, BN, 1), lambda i: (0, 0, i, 0)),
        pl.BlockSpec((BN, D), lambda i: (i, 0)),
    ],
    out_specs=[
        pl.BlockSpec((BN, D), lambda i: (i, 0)),
        pl.BlockSpec((BN, 1), lambda i: (i, 0)),
        pl.BlockSpec((BN, 1), lambda i: (i, 0)),
    ],
    out_shape=[
        jax.ShapeDtypeStruct((N, D), jnp.float32),
        jax.ShapeDtypeStruct((N, 1), jnp.float32),
        jax.ShapeDtypeStruct((N, 1), jnp.float32),
    ],
)


def _mid_body(p_ref, nd_ref, ns_ref, w_ref, b_ref, o_ref):
    agg = (p_ref[0] + p_ref[1]) * nd_ref[...]
    z = jnp.dot(agg, w_ref[...], preferred_element_type=jnp.float32)
    z = jnp.maximum(z + b_ref[...], 0.0)
    o_ref[...] = z * ns_ref[...]


_mid_call = pl.pallas_call(
    _mid_body,
    grid=(GRID,),
    in_specs=[
        pl.BlockSpec((NC, BN, D), lambda i: (0, i, 0)),
        pl.BlockSpec((BN, 1), lambda i: (i, 0)),
        pl.BlockSpec((BN, 1), lambda i: (i, 0)),
        pl.BlockSpec((D, D), lambda i: (0, 0)),
        pl.BlockSpec((1, D), lambda i: (0, 0)),
    ],
    out_specs=pl.BlockSpec((BN, D), lambda i: (i, 0)),
    out_shape=jax.ShapeDtypeStruct((N, D), jnp.float32),
)


def _fin_body(p_ref, nd_ref, w_ref, b_ref, o_ref):
    agg = (p_ref[0] + p_ref[1]) * nd_ref[...]
    z = jnp.dot(agg, w_ref[...], preferred_element_type=jnp.float32)
    o_ref[...] = jnp.maximum(z + b_ref[...], 0.0)


_fin_call = pl.pallas_call(
    _fin_body,
    grid=(GRID,),
    in_specs=[
        pl.BlockSpec((NC, BN, D), lambda i: (0, i, 0)),
        pl.BlockSpec((BN, 1), lambda i: (i, 0)),
        pl.BlockSpec((D, D), lambda i: (0, 0)),
        pl.BlockSpec((1, D), lambda i: (0, 0)),
    ],
    out_specs=pl.BlockSpec((BN, D), lambda i: (i, 0)),
    out_shape=jax.ShapeDtypeStruct((N, D), jnp.float32),
)


def kernel(x, edge_index, W1, b1, W2, b2):
    src = edge_index[0]
    dst = edge_index[1]
    npad = EPAD - E
    iota = jnp.arange(npad, dtype=jnp.int32)
    src_s = jnp.concatenate([src, iota % 128])
    dst_s = jnp.concatenate([dst, N + iota % (NPAD - N)])
    src_d = jnp.concatenate([src, 2 * N + iota % 2048])
    dst_d = jnp.concatenate([dst + N, 2 * N + iota % 2048])

    degs = _deg_kernel(src_d, dst_d)
    xs, ns, nd = _prep_call(degs.reshape(NC, 2, N, 1), x)

    p1 = _seg_kernel(xs, src_s, dst_s)
    zs = _mid_call(p1, nd, ns, W1, b1.reshape(1, D))

    p2 = _seg_kernel(zs, src_s, dst_s)
    out = _fin_call(p2, nd, W2, b2.reshape(1, D))
    return out


# speedup vs baseline: 3.6010x; 1.0408x over previous
import functools

import jax
import jax.numpy as jnp
from jax import lax
from jax.experimental import pallas as pl
from jax.experimental.pallas import tpu as pltpu
from jax.experimental.pallas import tpu_sc as plsc

N = 10000
E = 320000
D = 128

NC = 2
NS = 16
NW = NC * NS
CH = 96
EPW = 10080
NCHUNK = EPW // CH
EPAD = NW * EPW + 4 * CH
NPAD = 10112
RPT = NPAD // NS
DCH = 128
DEPW = 10240
DNCHUNK = DEPW // DCH
DEPAD = NW * DEPW + 4 * DCH

BN = 1000
GRID = N // BN

_mesh = plsc.VectorSubcoreMesh(core_axis_name="c", subcore_axis_name="s")


@functools.partial(
    pl.kernel,
    mesh=_mesh,
    out_type=jax.ShapeDtypeStruct((NC * 2 * N,), jnp.float32),
    scratch_types=[
        pltpu.VMEM_SHARED((2 * N + 2048,), jnp.float32),
        pltpu.VMEM((3, DCH), jnp.int32),
        pltpu.VMEM((3, DCH), jnp.int32),
        pltpu.VMEM((DCH,), jnp.float32),
        pltpu.VMEM((2000,), jnp.float32),
        pltpu.SemaphoreType.DMA,
        pltpu.SemaphoreType.DMA,
        pltpu.SemaphoreType.DMA,
    ],
)
def _deg_kernel(src_hbm, dstoff_hbm, out_hbm,
                deg_sp, src_v, dst_v, ones_v, stage_v, s1sem, s2sem, ssem):
    c = lax.axis_index("c")
    s = lax.axis_index("s")
    wid = c * NS + s
    base = wid * DEPW

    pltpu.sync_copy(src_hbm.at[pl.ds(base, DCH)], src_v.at[0])
    pltpu.sync_copy(dstoff_hbm.at[pl.ds(base, DCH)], dst_v.at[0])
    off1 = pl.multiple_of(base + DCH, 8)
    pltpu.make_async_copy(src_hbm.at[pl.ds(off1, DCH)], src_v.at[1], s1sem).start()
    pltpu.make_async_copy(dstoff_hbm.at[pl.ds(off1, DCH)], dst_v.at[1], s2sem).start()

    @pl.when(s < 10)
    def _():
        def zb(i, carry):
            stage_v[pl.ds(i * 16, 16)] = jnp.zeros((16,), jnp.float32)
            return carry
        lax.fori_loop(0, 2000 // 16, zb, 0)
        off = pl.multiple_of(s * 2000, 8)
        pltpu.sync_copy(stage_v, deg_sp.at[pl.ds(off, 2000)])

    for j in range(DCH // 16):
        ones_v[pl.ds(j * 16, 16)] = jnp.full((16,), 1.0, jnp.float32)

    plsc.subcore_barrier()

    def body(i, carry):
        b3 = lax.rem(i, 3)
        b3n = lax.rem(i + 1, 3)
        b3p = lax.rem(i + 2, 3)
        offp = pl.multiple_of(base + (i + 2) * DCH, 8)

        @pl.when(i > 0)
        def _():
            pltpu.make_async_copy(
                ones_v, deg_sp.at[src_v.at[b3p]], ssem).wait()
            pltpu.make_async_copy(
                ones_v, deg_sp.at[dst_v.at[b3p]], ssem).wait()

        pltpu.make_async_copy(
            src_hbm.at[pl.ds(offp, DCH)], src_v.at[b3], s1sem).wait()
        pltpu.make_async_copy(
            dstoff_hbm.at[pl.ds(offp, DCH)], dst_v.at[b3], s2sem).wait()
        pltpu.make_async_copy(
            ones_v, deg_sp.at[src_v.at[b3]], ssem).start(add=True)
        pltpu.make_async_copy(
            ones_v, deg_sp.at[dst_v.at[b3]], ssem).start(add=True)
        pltpu.make_async_copy(
            src_hbm.at[pl.ds(offp, DCH)], src_v.at[b3p], s1sem).start()
        pltpu.make_async_copy(
            dstoff_hbm.at[pl.ds(offp, DCH)], dst_v.at[b3p], s2sem).start()
        return carry

    lax.fori_loop(0, DNCHUNK, body, 0)
    pltpu.make_async_copy(
        ones_v, deg_sp.at[src_v.at[(DNCHUNK - 1) % 3]], ssem).wait()
    pltpu.make_async_copy(
        ones_v, deg_sp.at[dst_v.at[(DNCHUNK - 1) % 3]], ssem).wait()
    pltpu.make_async_copy(
        src_hbm.at[pl.ds(base, DCH)], src_v.at[0], s1sem).wait()
    pltpu.make_async_copy(
        dstoff_hbm.at[pl.ds(base, DCH)], dst_v.at[0], s2sem).wait()
    plsc.subcore_barrier()

    @pl.when(s < 10)
    def _():
        off = pl.multiple_of(s * 2000, 8)
        pltpu.sync_copy(deg_sp.at[pl.ds(off, 2000)], stage_v)
        pltpu.sync_copy(stage_v, out_hbm.at[pl.ds(c * 2 * N + off, 2000)])


@functools.partial(
    pl.kernel,
    mesh=_mesh,
    out_type=jax.ShapeDtypeStruct((NC, NPAD, D), jnp.float32),
    scratch_types=[
        pltpu.VMEM_SHARED((NPAD, D), jnp.float32),
        pltpu.VMEM((4, CH), jnp.int32),
        pltpu.VMEM((4, CH), jnp.int32),
        pltpu.VMEM((3, CH, D), jnp.float32),
        pltpu.SemaphoreType.DMA,
        pltpu.SemaphoreType.DMA,
        pltpu.SemaphoreType.DMA,
        pltpu.SemaphoreType.DMA,
        pltpu.SemaphoreType.DMA,
    ],
)
def _seg_kernel(xs_hbm, src_hbm, dst_hbm, out_hbm,
                agg_sp, src_v, dst_v, rows_v, ga, gb, s1sem, s2sem, ssem):
    c = lax.axis_index("c")
    s = lax.axis_index("s")
    wid = c * NS + s
    base = wid * EPW

    pltpu.sync_copy(src_hbm.at[pl.ds(base, CH)], src_v.at[0])
    pltpu.sync_copy(dst_hbm.at[pl.ds(base, CH)], dst_v.at[0])
    pltpu.make_async_copy(xs_hbm.at[src_v.at[0]], rows_v.at[0], ga).start()

    def _off(k):
        return pl.multiple_of(base + k * CH, 8)

    pltpu.make_async_copy(src_hbm.at[pl.ds(_off(1), CH)], src_v.at[1], s1sem).start()
    pltpu.make_async_copy(dst_hbm.at[pl.ds(_off(1), CH)], dst_v.at[1], s2sem).start()
    pltpu.make_async_copy(src_hbm.at[pl.ds(_off(1), CH)], src_v.at[1], s1sem).wait()
    pltpu.make_async_copy(xs_hbm.at[src_v.at[1]], rows_v.at[1], gb).start()
    pltpu.make_async_copy(src_hbm.at[pl.ds(_off(2), CH)], src_v.at[2], s1sem).start()
    pltpu.make_async_copy(src_hbm.at[pl.ds(_off(3), CH)], src_v.at[3], s1sem).start()

    def zb(i, carry):
        for j in range(D // 16):
            rows_v[2, i, pl.ds(j * 16, 16)] = jnp.zeros((16,), jnp.float32)
        return carry
    lax.fori_loop(0, CH, zb, 0)
    roff = pl.multiple_of(s * RPT, 8)
    for k in range(RPT // CH):
        pltpu.make_async_copy(
            rows_v.at[2], agg_sp.at[pl.ds(roff + k * CH, CH)], ssem).start()
    pltpu.make_async_copy(
        rows_v.at[2, pl.ds(0, RPT % CH)],
        agg_sp.at[pl.ds(roff + (RPT // CH) * CH, RPT % CH)], ssem).start()
    for k in range(RPT // CH):
        pltpu.make_async_copy(
            rows_v.at[2], agg_sp.at[pl.ds(roff + k * CH, CH)], ssem).wait()
    pltpu.make_async_copy(
        rows_v.at[2, pl.ds(0, RPT % CH)],
        agg_sp.at[pl.ds(roff + (RPT // CH) * CH, RPT % CH)], ssem).wait()

    plsc.subcore_barrier()

    def body(i, carry):
        b2 = lax.rem(i, 2)
        b3 = lax.rem(i, 3)
        b3p2 = lax.rem(i + 2, 3)
        b4 = lax.rem(i, 4)
        b4p2 = lax.rem(i + 2, 4)
        offp2 = pl.multiple_of(base + (i + 2) * CH, 8)
        offp4 = pl.multiple_of(base + (i + 4) * CH, 8)

        @pl.when(b2 == 0)
        def _():
            pltpu.make_async_copy(
                xs_hbm.at[src_v.at[b4]], rows_v.at[b3], ga).wait()

        @pl.when(b2 == 1)
        def _():
            pltpu.make_async_copy(
                xs_hbm.at[src_v.at[b4]], rows_v.at[b3], gb).wait()

        @pl.when(i > 0)
        def _():
            pltpu.make_async_copy(
                rows_v.at[b3p2], agg_sp.at[dst_v.at[b4p2]], ssem).wait()

        @pl.when(i == 0)
        def _():
            pltpu.make_async_copy(
                src_hbm.at[pl.ds(offp2, CH)], src_v.at[b4p2], s1sem).wait()

        pltpu.make_async_copy(
            src_hbm.at[pl.ds(offp2, CH)], src_v.at[b4p2], s1sem).wait()

        @pl.when(b2 == 0)
        def _():
            pltpu.make_async_copy(
                xs_hbm.at[src_v.at[b4p2]], rows_v.at[b3p2], ga).start()

        @pl.when(b2 == 1)
        def _():
            pltpu.make_async_copy(
                xs_hbm.at[src_v.at[b4p2]], rows_v.at[b3p2], gb).start()

        pltpu.make_async_copy(
            src_hbm.at[pl.ds(offp4, CH)], src_v.at[b4], s1sem).start()
        pltpu.make_async_copy(
            dst_hbm.at[pl.ds(offp2, CH)], dst_v.at[b4p2], s2sem).wait()
        pltpu.make_async_copy(
            rows_v.at[b3], agg_sp.at[dst_v.at[b4]], ssem).start(add=True)
        pltpu.make_async_copy(
            dst_hbm.at[pl.ds(offp2, CH)], dst_v.at[b4p2], s2sem).start()
        return carry

    lax.fori_loop(0, NCHUNK, body, 0)
    pltpu.make_async_copy(
        xs_hbm.at[src_v.at[NCHUNK % 4]], rows_v.at[NCHUNK % 3], ga).wait()
    pltpu.make_async_copy(
        xs_hbm.at[src_v.at[(NCHUNK + 1) % 4]],
        rows_v.at[(NCHUNK + 1) % 3], gb).wait()
    pltpu.make_async_copy(
        rows_v.at[(NCHUNK - 1) % 3],
        agg_sp.at[dst_v.at[(NCHUNK - 1) % 4]], ssem).wait()
    pltpu.make_async_copy(
        src_hbm.at[pl.ds(base, CH)], src_v.at[0], s1sem).wait()
    pltpu.make_async_copy(
        dst_hbm.at[pl.ds(base, CH)], dst_v.at[0], s2sem).wait()
    plsc.subcore_barrier()

    droff = pl.multiple_of(s * RPT, 8)
    nfull = RPT // CH
    sizes = [CH] * nfull + [RPT % CH]

    def _stage(k):
        return rows_v.at[k % 2, pl.ds(0, sizes[k])]

    def _wr(k):
        off = pl.multiple_of(droff + k * CH, 8)
        return pltpu.make_async_copy(
            _stage(k), out_hbm.at[c, pl.ds(off, sizes[k])],
            ga if k % 2 == 0 else gb)

    for k in range(nfull + 1):
        if k >= 2:
            _wr(k - 2).wait()
        off = pl.multiple_of(droff + k * CH, 8)
        pltpu.sync_copy(agg_sp.at[pl.ds(off, sizes[k])], _stage(k))
        _wr(k).start()
    _wr(nfull - 1).wait()
    _wr(nfull).wait()


def _prep_body(deg_ref, x_ref, xs_ref, ns_ref, nd_ref):
    d = deg_ref[...]
    deg_o = d[0, 0] + d[1, 0]
    deg_i = d[0, 1] + d[1, 1]
    ns = lax.rsqrt(jnp.maximum(deg_o, 1.0))
    nd = lax.rsqrt(jnp.maximum(deg_i, 1.0))
    xs_ref[...] = x_ref[...] * ns
    ns_ref[...] = ns
    nd_ref[...] = nd


_prep_call = pl.pallas_call(
    _prep_body,
    grid=(GRID,),
    in_specs=[
        pl.BlockSpec((NC, 2, BN, 1), lambda i: (0, 0, i, 0)),
        pl.BlockSpec((BN, D), lambda i: (i, 0)),
    ],
    out_specs=[
        pl.BlockSpec((BN, D), lambda i: (i, 0)),
        pl.BlockSpec((BN, 1), lambda i: (i, 0)),
        pl.BlockSpec((BN, 1), lambda i: (i, 0)),
    ],
    out_shape=[
        jax.ShapeDtypeStruct((N, D), jnp.float32),
        jax.ShapeDtypeStruct((N, 1), jnp.float32),
        jax.ShapeDtypeStruct((N, 1), jnp.float32),
    ],
)


def _mid_body(p_ref, nd_ref, ns_ref, w_ref, b_ref, o_ref):
    agg = (p_ref[0] + p_ref[1]) * nd_ref[...]
    z = jnp.dot(agg, w_ref[...], preferred_element_type=jnp.float32)
    z = jnp.maximum(z + b_ref[...], 0.0)
    o_ref[...] = z * ns_ref[...]


_mid_call = pl.pallas_call(
    _mid_body,
    grid=(GRID,),
    in_specs=[
        pl.BlockSpec((NC, BN, D), lambda i: (0, i, 0)),
        pl.BlockSpec((BN, 1), lambda i: (i, 0)),
        pl.BlockSpec((BN, 1), lambda i: (i, 0)),
        pl.BlockSpec((D, D), lambda i: (0, 0)),
        pl.BlockSpec((1, D), lambda i: (0, 0)),
    ],
    out_specs=pl.BlockSpec((BN, D), lambda i: (i, 0)),
    out_shape=jax.ShapeDtypeStruct((N, D), jnp.float32),
)


def _fin_body(p_ref, nd_ref, w_ref, b_ref, o_ref):
    agg = (p_ref[0] + p_ref[1]) * nd_ref[...]
    z = jnp.dot(agg, w_ref[...], preferred_element_type=jnp.float32)
    o_ref[...] = jnp.maximum(z + b_ref[...], 0.0)


_fin_call = pl.pallas_call(
    _fin_body,
    grid=(GRID,),
    in_specs=[
        pl.BlockSpec((NC, BN, D), lambda i: (0, i, 0)),
        pl.BlockSpec((BN, 1), lambda i: (i, 0)),
        pl.BlockSpec((D, D), lambda i: (0, 0)),
        pl.BlockSpec((1, D), lambda i: (0, 0)),
    ],
    out_specs=pl.BlockSpec((BN, D), lambda i: (i, 0)),
    out_shape=jax.ShapeDtypeStruct((N, D), jnp.float32),
)


def kernel(x, edge_index, W1, b1, W2, b2):
    src = edge_index[0]
    dst = edge_index[1]
    npad = EPAD - E
    iota = jnp.arange(npad, dtype=jnp.int32)
    src_s = jnp.concatenate([src, iota % 128])
    dst_s = jnp.concatenate([dst, N + iota % (NPAD - N)])
    diota = jnp.arange(DEPAD - E, dtype=jnp.int32)
    src_d = jnp.concatenate([src, 2 * N + diota % 2048])
    dst_d = jnp.concatenate([dst + N, 2 * N + diota % 2048])

    degs = _deg_kernel(src_d, dst_d)
    xs, ns, nd = _prep_call(degs.reshape(NC, 2, N, 1), x)

    p1 = _seg_kernel(xs, src_s, dst_s)
    zs = _mid_call(p1, nd, ns, W1, b1.reshape(1, D))

    p2 = _seg_kernel(zs, src_s, dst_s)
    out = _fin_call(p2, nd, W2, b2.reshape(1, D))
    return out


# speedup vs baseline: 3.6399x; 1.0108x over previous
import functools

import jax
import jax.numpy as jnp
from jax import lax
from jax.experimental import pallas as pl
from jax.experimental.pallas import tpu as pltpu
from jax.experimental.pallas import tpu_sc as plsc

N = 10000
E = 320000
D = 128

NC = 2
NS = 16
NW = NC * NS
CH = 128
EPW = 10240
NCHUNK = EPW // CH
EPAD = NW * EPW + 4 * CH
NPAD = 10112
RPT = NPAD // NS
DCH = 128
DEPW = 10240
DNCHUNK = DEPW // DCH
DEPAD = NW * DEPW + 4 * DCH

BN = 1000
GRID = N // BN

_mesh = plsc.VectorSubcoreMesh(core_axis_name="c", subcore_axis_name="s")


@functools.partial(
    pl.kernel,
    mesh=_mesh,
    out_type=jax.ShapeDtypeStruct((NC * 2 * N,), jnp.float32),
    scratch_types=[
        pltpu.VMEM_SHARED((2 * N + 2048,), jnp.float32),
        pltpu.VMEM((3, DCH), jnp.int32),
        pltpu.VMEM((3, DCH), jnp.int32),
        pltpu.VMEM((DCH,), jnp.float32),
        pltpu.VMEM((2000,), jnp.float32),
        pltpu.SemaphoreType.DMA,
        pltpu.SemaphoreType.DMA,
        pltpu.SemaphoreType.DMA,
    ],
)
def _deg_kernel(src_hbm, dstoff_hbm, out_hbm,
                deg_sp, src_v, dst_v, ones_v, stage_v, s1sem, s2sem, ssem):
    c = lax.axis_index("c")
    s = lax.axis_index("s")
    wid = c * NS + s
    base = wid * DEPW

    pltpu.sync_copy(src_hbm.at[pl.ds(base, DCH)], src_v.at[0])
    pltpu.sync_copy(dstoff_hbm.at[pl.ds(base, DCH)], dst_v.at[0])
    off1 = pl.multiple_of(base + DCH, 8)
    pltpu.make_async_copy(src_hbm.at[pl.ds(off1, DCH)], src_v.at[1], s1sem).start()
    pltpu.make_async_copy(dstoff_hbm.at[pl.ds(off1, DCH)], dst_v.at[1], s2sem).start()

    @pl.when(s < 10)
    def _():
        def zb(i, carry):
            stage_v[pl.ds(i * 16, 16)] = jnp.zeros((16,), jnp.float32)
            return carry
        lax.fori_loop(0, 2000 // 16, zb, 0)
        off = pl.multiple_of(s * 2000, 8)
        pltpu.sync_copy(stage_v, deg_sp.at[pl.ds(off, 2000)])

    for j in range(DCH // 16):
        ones_v[pl.ds(j * 16, 16)] = jnp.full((16,), 1.0, jnp.float32)

    plsc.subcore_barrier()

    def body(i, carry):
        b3 = lax.rem(i, 3)
        b3n = lax.rem(i + 1, 3)
        b3p = lax.rem(i + 2, 3)
        offp = pl.multiple_of(base + (i + 2) * DCH, 8)

        @pl.when(i > 0)
        def _():
            pltpu.make_async_copy(
                ones_v, deg_sp.at[src_v.at[b3p]], ssem).wait()
            pltpu.make_async_copy(
                ones_v, deg_sp.at[dst_v.at[b3p]], ssem).wait()

        pltpu.make_async_copy(
            src_hbm.at[pl.ds(offp, DCH)], src_v.at[b3], s1sem).wait()
        pltpu.make_async_copy(
            dstoff_hbm.at[pl.ds(offp, DCH)], dst_v.at[b3], s2sem).wait()
        pltpu.make_async_copy(
            ones_v, deg_sp.at[src_v.at[b3]], ssem).start(add=True)
        pltpu.make_async_copy(
            ones_v, deg_sp.at[dst_v.at[b3]], ssem).start(add=True)
        pltpu.make_async_copy(
            src_hbm.at[pl.ds(offp, DCH)], src_v.at[b3p], s1sem).start()
        pltpu.make_async_copy(
            dstoff_hbm.at[pl.ds(offp, DCH)], dst_v.at[b3p], s2sem).start()
        return carry

    lax.fori_loop(0, DNCHUNK, body, 0)
    pltpu.make_async_copy(
        ones_v, deg_sp.at[src_v.at[(DNCHUNK - 1) % 3]], ssem).wait()
    pltpu.make_async_copy(
        ones_v, deg_sp.at[dst_v.at[(DNCHUNK - 1) % 3]], ssem).wait()
    pltpu.make_async_copy(
        src_hbm.at[pl.ds(base, DCH)], src_v.at[0], s1sem).wait()
    pltpu.make_async_copy(
        dstoff_hbm.at[pl.ds(base, DCH)], dst_v.at[0], s2sem).wait()
    plsc.subcore_barrier()

    @pl.when(s < 10)
    def _():
        off = pl.multiple_of(s * 2000, 8)
        pltpu.sync_copy(deg_sp.at[pl.ds(off, 2000)], stage_v)
        pltpu.sync_copy(stage_v, out_hbm.at[pl.ds(c * 2 * N + off, 2000)])


@functools.partial(
    pl.kernel,
    mesh=_mesh,
    out_type=jax.ShapeDtypeStruct((NC, NPAD, D), jnp.float32),
    scratch_types=[
        pltpu.VMEM_SHARED((NPAD, D), jnp.float32),
        pltpu.VMEM((4, CH), jnp.int32),
        pltpu.VMEM((3, CH), jnp.int32),
        pltpu.VMEM((3, CH, D), jnp.float32),
        pltpu.SemaphoreType.DMA,
        pltpu.SemaphoreType.DMA,
        pltpu.SemaphoreType.DMA,
        pltpu.SemaphoreType.DMA,
        pltpu.SemaphoreType.DMA,
    ],
)
def _seg_kernel(xs_hbm, src_hbm, dst_hbm, out_hbm,
                agg_sp, src_v, dst_v, rows_v, ga, gb, s1sem, s2sem, ssem):
    c = lax.axis_index("c")
    s = lax.axis_index("s")
    wid = c * NS + s
    base = wid * EPW

    pltpu.sync_copy(src_hbm.at[pl.ds(base, CH)], src_v.at[0])
    pltpu.sync_copy(dst_hbm.at[pl.ds(base, CH)], dst_v.at[0])
    pltpu.make_async_copy(xs_hbm.at[src_v.at[0]], rows_v.at[0], ga).start()

    def _off(k):
        return pl.multiple_of(base + k * CH, 8)

    pltpu.make_async_copy(src_hbm.at[pl.ds(_off(1), CH)], src_v.at[1], s1sem).start()
    pltpu.make_async_copy(dst_hbm.at[pl.ds(_off(1), CH)], dst_v.at[1], s2sem).start()
    pltpu.make_async_copy(src_hbm.at[pl.ds(_off(1), CH)], src_v.at[1], s1sem).wait()
    pltpu.make_async_copy(xs_hbm.at[src_v.at[1]], rows_v.at[1], gb).start()
    pltpu.make_async_copy(src_hbm.at[pl.ds(_off(2), CH)], src_v.at[2], s1sem).start()
    pltpu.make_async_copy(src_hbm.at[pl.ds(_off(3), CH)], src_v.at[3], s1sem).start()

    def zb(i, carry):
        for j in range(D // 16):
            rows_v[2, i, pl.ds(j * 16, 16)] = jnp.zeros((16,), jnp.float32)
        return carry
    lax.fori_loop(0, CH, zb, 0)
    roff = pl.multiple_of(s * RPT, 8)
    for k in range(RPT // CH):
        pltpu.make_async_copy(
            rows_v.at[2], agg_sp.at[pl.ds(roff + k * CH, CH)], ssem).start()
    pltpu.make_async_copy(
        rows_v.at[2, pl.ds(0, RPT % CH)],
        agg_sp.at[pl.ds(roff + (RPT // CH) * CH, RPT % CH)], ssem).start()
    for k in range(RPT // CH):
        pltpu.make_async_copy(
            rows_v.at[2], agg_sp.at[pl.ds(roff + k * CH, CH)], ssem).wait()
    pltpu.make_async_copy(
        rows_v.at[2, pl.ds(0, RPT % CH)],
        agg_sp.at[pl.ds(roff + (RPT // CH) * CH, RPT % CH)], ssem).wait()

    plsc.subcore_barrier()

    def body(i, carry):
        b2 = lax.rem(i, 2)
        b3 = lax.rem(i, 3)
        b3p2 = lax.rem(i + 2, 3)
        b4 = lax.rem(i, 4)
        b4p2 = lax.rem(i + 2, 4)
        offp2 = pl.multiple_of(base + (i + 2) * CH, 8)
        offp4 = pl.multiple_of(base + (i + 4) * CH, 8)

        @pl.when(b2 == 0)
        def _():
            pltpu.make_async_copy(
                xs_hbm.at[src_v.at[b4]], rows_v.at[b3], ga).wait()

        @pl.when(b2 == 1)
        def _():
            pltpu.make_async_copy(
                xs_hbm.at[src_v.at[b4]], rows_v.at[b3], gb).wait()

        @pl.when(i > 0)
        def _():
            pltpu.make_async_copy(
                rows_v.at[b3p2], agg_sp.at[dst_v.at[b3p2]], ssem).wait()

        @pl.when(i == 0)
        def _():
            pltpu.make_async_copy(
                src_hbm.at[pl.ds(offp2, CH)], src_v.at[b4p2], s1sem).wait()

        pltpu.make_async_copy(
            src_hbm.at[pl.ds(offp2, CH)], src_v.at[b4p2], s1sem).wait()

        @pl.when(b2 == 0)
        def _():
            pltpu.make_async_copy(
                xs_hbm.at[src_v.at[b4p2]], rows_v.at[b3p2], ga).start()

        @pl.when(b2 == 1)
        def _():
            pltpu.make_async_copy(
                xs_hbm.at[src_v.at[b4p2]], rows_v.at[b3p2], gb).start()

        pltpu.make_async_copy(
            src_hbm.at[pl.ds(offp4, CH)], src_v.at[b4], s1sem).start()
        pltpu.make_async_copy(
            dst_hbm.at[pl.ds(offp2, CH)], dst_v.at[b3p2], s2sem).wait()
        pltpu.make_async_copy(
            rows_v.at[b3], agg_sp.at[dst_v.at[b3]], ssem).start(add=True)
        pltpu.make_async_copy(
            dst_hbm.at[pl.ds(offp2, CH)], dst_v.at[b3p2], s2sem).start()
        return carry

    lax.fori_loop(0, NCHUNK, body, 0)
    pltpu.make_async_copy(
        xs_hbm.at[src_v.at[NCHUNK % 4]], rows_v.at[NCHUNK % 3], ga).wait()
    pltpu.make_async_copy(
        xs_hbm.at[src_v.at[(NCHUNK + 1) % 4]],
        rows_v.at[(NCHUNK + 1) % 3], gb).wait()
    pltpu.make_async_copy(
        rows_v.at[(NCHUNK - 1) % 3],
        agg_sp.at[dst_v.at[(NCHUNK - 1) % 3]], ssem).wait()
    pltpu.make_async_copy(
        src_hbm.at[pl.ds(base, CH)], src_v.at[0], s1sem).wait()
    pltpu.make_async_copy(
        dst_hbm.at[pl.ds(base, CH)], dst_v.at[0], s2sem).wait()
    plsc.subcore_barrier()

    droff = pl.multiple_of(s * RPT, 8)
    nfull = RPT // CH
    sizes = [CH] * nfull + [RPT % CH]

    def _stage(k):
        return rows_v.at[k % 2, pl.ds(0, sizes[k])]

    def _wr(k):
        off = pl.multiple_of(droff + k * CH, 8)
        return pltpu.make_async_copy(
            _stage(k), out_hbm.at[c, pl.ds(off, sizes[k])],
            ga if k % 2 == 0 else gb)

    for k in range(nfull + 1):
        if k >= 2:
            _wr(k - 2).wait()
        off = pl.multiple_of(droff + k * CH, 8)
        pltpu.sync_copy(agg_sp.at[pl.ds(off, sizes[k])], _stage(k))
        _wr(k).start()
    _wr(nfull - 1).wait()
    _wr(nfull).wait()


def _prep_body(deg_ref, x_ref, xs_ref, ns_ref, nd_ref):
    d = deg_ref[...]
    deg_o = d[0, 0] + d[1, 0]
    deg_i = d[0, 1] + d[1, 1]
    ns = lax.rsqrt(jnp.maximum(deg_o, 1.0))
    nd = lax.rsqrt(jnp.maximum(deg_i, 1.0))
    xs_ref[...] = x_ref[...] * ns
    ns_ref[...] = ns
    nd_ref[...] = nd


_prep_call = pl.pallas_call(
    _prep_body,
    grid=(GRID,),
    in_specs=[
        pl.BlockSpec((NC, 2, BN, 1), lambda i: (0, 0, i, 0)),
        pl.BlockSpec((BN, D), lambda i: (i, 0)),
    ],
    out_specs=[
        pl.BlockSpec((BN, D), lambda i: (i, 0)),
        pl.BlockSpec((BN, 1), lambda i: (i, 0)),
        pl.BlockSpec((BN, 1), lambda i: (i, 0)),
    ],
    out_shape=[
        jax.ShapeDtypeStruct((N, D), jnp.float32),
        jax.ShapeDtypeStruct((N, 1), jnp.float32),
        jax.ShapeDtypeStruct((N, 1), jnp.float32),
    ],
)


def _mid_body(p_ref, nd_ref, ns_ref, w_ref, b_ref, o_ref):
    agg = (p_ref[0] + p_ref[1]) * nd_ref[...]
    z = jnp.dot(agg, w_ref[...], preferred_element_type=jnp.float32)
    z = jnp.maximum(z + b_ref[...], 0.0)
    o_ref[...] = z * ns_ref[...]


_mid_call = pl.pallas_call(
    _mid_body,
    grid=(GRID,),
    in_specs=[
        pl.BlockSpec((NC, BN, D), lambda i: (0, i, 0)),
        pl.BlockSpec((BN, 1), lambda i: (i, 0)),
        pl.BlockSpec((BN, 1), lambda i: (i, 0)),
        pl.BlockSpec((D, D), lambda i: (0, 0)),
        pl.BlockSpec((1, D), lambda i: (0, 0)),
    ],
    out_specs=pl.BlockSpec((BN, D), lambda i: (i, 0)),
    out_shape=jax.ShapeDtypeStruct((N, D), jnp.float32),
)


def _fin_body(p_ref, nd_ref, w_ref, b_ref, o_ref):
    agg = (p_ref[0] + p_ref[1]) * nd_ref[...]
    z = jnp.dot(agg, w_ref[...], preferred_element_type=jnp.float32)
    o_ref[...] = jnp.maximum(z + b_ref[...], 0.0)


_fin_call = pl.pallas_call(
    _fin_body,
    grid=(GRID,),
    in_specs=[
        pl.BlockSpec((NC, BN, D), lambda i: (0, i, 0)),
        pl.BlockSpec((BN, 1), lambda i: (i, 0)),
        pl.BlockSpec((D, D), lambda i: (0, 0)),
        pl.BlockSpec((1, D), lambda i: (0, 0)),
    ],
    out_specs=pl.BlockSpec((BN, D), lambda i: (i, 0)),
    out_shape=jax.ShapeDtypeStruct((N, D), jnp.float32),
)


def kernel(x, edge_index, W1, b1, W2, b2):
    src = edge_index[0]
    dst = edge_index[1]
    npad = EPAD - E
    iota = jnp.arange(npad, dtype=jnp.int32)
    src_s = jnp.concatenate([src, iota % 128])
    dst_s = jnp.concatenate([dst, N + iota % (NPAD - N)])
    diota = jnp.arange(DEPAD - E, dtype=jnp.int32)
    src_d = jnp.concatenate([src, 2 * N + diota % 2048])
    dst_d = jnp.concatenate([dst + N, 2 * N + diota % 2048])

    degs = _deg_kernel(src_d, dst_d)
    xs, ns, nd = _prep_call(degs.reshape(NC, 2, N, 1), x)

    p1 = _seg_kernel(xs, src_s, dst_s)
    zs = _mid_call(p1, nd, ns, W1, b1.reshape(1, D))

    p2 = _seg_kernel(zs, src_s, dst_s)
    out = _fin_call(p2, nd, W2, b2.reshape(1, D))
    return out


# speedup vs baseline: 3.7134x; 1.0202x over previous
import functools

import jax
import jax.numpy as jnp
from jax import lax
from jax.experimental import pallas as pl
from jax.experimental.pallas import tpu as pltpu
from jax.experimental.pallas import tpu_sc as plsc

N = 10000
E = 320000
D = 128

NC = 2
NS = 16
NW = NC * NS
CH = 128
EPW = 10240
NCHUNK = EPW // CH
EPAD = NW * EPW + 4 * CH
NPAD = 10112
RPT = NPAD // NS
DCH = 128
DEPW = 10240
DNCHUNK = DEPW // DCH
DEPAD = NW * DEPW + 4 * DCH

BN = 2000
GRID = N // BN

_mesh = plsc.VectorSubcoreMesh(core_axis_name="c", subcore_axis_name="s")


@functools.partial(
    pl.kernel,
    mesh=_mesh,
    out_type=jax.ShapeDtypeStruct((NC * 2 * N,), jnp.float32),
    scratch_types=[
        pltpu.VMEM_SHARED((2 * N + 2048,), jnp.float32),
        pltpu.VMEM((3, DCH), jnp.int32),
        pltpu.VMEM((3, DCH), jnp.int32),
        pltpu.VMEM((DCH,), jnp.float32),
        pltpu.VMEM((2000,), jnp.float32),
        pltpu.SemaphoreType.DMA,
        pltpu.SemaphoreType.DMA,
        pltpu.SemaphoreType.DMA,
    ],
)
def _deg_kernel(src_hbm, dstoff_hbm, out_hbm,
                deg_sp, src_v, dst_v, ones_v, stage_v, s1sem, s2sem, ssem):
    c = lax.axis_index("c")
    s = lax.axis_index("s")
    wid = c * NS + s
    base = wid * DEPW

    pltpu.sync_copy(src_hbm.at[pl.ds(base, DCH)], src_v.at[0])
    pltpu.sync_copy(dstoff_hbm.at[pl.ds(base, DCH)], dst_v.at[0])
    off1 = pl.multiple_of(base + DCH, 8)
    pltpu.make_async_copy(src_hbm.at[pl.ds(off1, DCH)], src_v.at[1], s1sem).start()
    pltpu.make_async_copy(dstoff_hbm.at[pl.ds(off1, DCH)], dst_v.at[1], s2sem).start()

    @pl.when(s < 10)
    def _():
        def zb(i, carry):
            stage_v[pl.ds(i * 16, 16)] = jnp.zeros((16,), jnp.float32)
            return carry
        lax.fori_loop(0, 2000 // 16, zb, 0)
        off = pl.multiple_of(s * 2000, 8)
        pltpu.sync_copy(stage_v, deg_sp.at[pl.ds(off, 2000)])

    for j in range(DCH // 16):
        ones_v[pl.ds(j * 16, 16)] = jnp.full((16,), 1.0, jnp.float32)

    plsc.subcore_barrier()

    def body(i, carry):
        b3 = lax.rem(i, 3)
        b3n = lax.rem(i + 1, 3)
        b3p = lax.rem(i + 2, 3)
        offp = pl.multiple_of(base + (i + 2) * DCH, 8)

        @pl.when(i > 0)
        def _():
            pltpu.make_async_copy(
                ones_v, deg_sp.at[src_v.at[b3p]], ssem).wait()
            pltpu.make_async_copy(
                ones_v, deg_sp.at[dst_v.at[b3p]], ssem).wait()

        pltpu.make_async_copy(
            src_hbm.at[pl.ds(offp, DCH)], src_v.at[b3], s1sem).wait()
        pltpu.make_async_copy(
            dstoff_hbm.at[pl.ds(offp, DCH)], dst_v.at[b3], s2sem).wait()
        pltpu.make_async_copy(
            ones_v, deg_sp.at[src_v.at[b3]], ssem).start(add=True)
        pltpu.make_async_copy(
            ones_v, deg_sp.at[dst_v.at[b3]], ssem).start(add=True)
        pltpu.make_async_copy(
            src_hbm.at[pl.ds(offp, DCH)], src_v.at[b3p], s1sem).start()
        pltpu.make_async_copy(
            dstoff_hbm.at[pl.ds(offp, DCH)], dst_v.at[b3p], s2sem).start()
        return carry

    lax.fori_loop(0, DNCHUNK, body, 0)
    pltpu.make_async_copy(
        ones_v, deg_sp.at[src_v.at[(DNCHUNK - 1) % 3]], ssem).wait()
    pltpu.make_async_copy(
        ones_v, deg_sp.at[dst_v.at[(DNCHUNK - 1) % 3]], ssem).wait()
    pltpu.make_async_copy(
        src_hbm.at[pl.ds(base, DCH)], src_v.at[0], s1sem).wait()
    pltpu.make_async_copy(
        dstoff_hbm.at[pl.ds(base, DCH)], dst_v.at[0], s2sem).wait()
    plsc.subcore_barrier()

    @pl.when(s < 10)
    def _():
        off = pl.multiple_of(s * 2000, 8)
        pltpu.sync_copy(deg_sp.at[pl.ds(off, 2000)], stage_v)
        pltpu.sync_copy(stage_v, out_hbm.at[pl.ds(c * 2 * N + off, 2000)])


@functools.partial(
    pl.kernel,
    mesh=_mesh,
    out_type=jax.ShapeDtypeStruct((NC, NPAD, D), jnp.float32),
    scratch_types=[
        pltpu.VMEM_SHARED((NPAD, D), jnp.float32),
        pltpu.VMEM((4, CH), jnp.int32),
        pltpu.VMEM((3, CH), jnp.int32),
        pltpu.VMEM((3, CH, D), jnp.float32),
        pltpu.SemaphoreType.DMA,
        pltpu.SemaphoreType.DMA,
        pltpu.SemaphoreType.DMA,
        pltpu.SemaphoreType.DMA,
        pltpu.SemaphoreType.DMA,
    ],
)
def _seg_kernel(xs_hbm, src_hbm, dst_hbm, out_hbm,
                agg_sp, src_v, dst_v, rows_v, ga, gb, s1sem, s2sem, ssem):
    c = lax.axis_index("c")
    s = lax.axis_index("s")
    wid = c * NS + s
    base = wid * EPW

    pltpu.sync_copy(src_hbm.at[pl.ds(base, CH)], src_v.at[0])
    pltpu.sync_copy(dst_hbm.at[pl.ds(base, CH)], dst_v.at[0])
    pltpu.make_async_copy(xs_hbm.at[src_v.at[0]], rows_v.at[0], ga).start()

    def _off(k):
        return pl.multiple_of(base + k * CH, 8)

    pltpu.make_async_copy(src_hbm.at[pl.ds(_off(1), CH)], src_v.at[1], s1sem).start()
    pltpu.make_async_copy(dst_hbm.at[pl.ds(_off(1), CH)], dst_v.at[1], s2sem).start()
    pltpu.make_async_copy(src_hbm.at[pl.ds(_off(1), CH)], src_v.at[1], s1sem).wait()
    pltpu.make_async_copy(xs_hbm.at[src_v.at[1]], rows_v.at[1], gb).start()
    pltpu.make_async_copy(src_hbm.at[pl.ds(_off(2), CH)], src_v.at[2], s1sem).start()
    pltpu.make_async_copy(src_hbm.at[pl.ds(_off(3), CH)], src_v.at[3], s1sem).start()

    def zb(i, carry):
        for j in range(D // 16):
            rows_v[2, i, pl.ds(j * 16, 16)] = jnp.zeros((16,), jnp.float32)
        return carry
    lax.fori_loop(0, CH, zb, 0)
    roff = pl.multiple_of(s * RPT, 8)
    for k in range(RPT // CH):
        pltpu.make_async_copy(
            rows_v.at[2], agg_sp.at[pl.ds(roff + k * CH, CH)], ssem).start()
    pltpu.make_async_copy(
        rows_v.at[2, pl.ds(0, RPT % CH)],
        agg_sp.at[pl.ds(roff + (RPT // CH) * CH, RPT % CH)], ssem).start()
    for k in range(RPT // CH):
        pltpu.make_async_copy(
            rows_v.at[2], agg_sp.at[pl.ds(roff + k * CH, CH)], ssem).wait()
    pltpu.make_async_copy(
        rows_v.at[2, pl.ds(0, RPT % CH)],
        agg_sp.at[pl.ds(roff + (RPT // CH) * CH, RPT % CH)], ssem).wait()

    plsc.subcore_barrier()

    def body(i, carry):
        b2 = lax.rem(i, 2)
        b3 = lax.rem(i, 3)
        b3p2 = lax.rem(i + 2, 3)
        b4 = lax.rem(i, 4)
        b4p2 = lax.rem(i + 2, 4)
        offp2 = pl.multiple_of(base + (i + 2) * CH, 8)
        offp4 = pl.multiple_of(base + (i + 4) * CH, 8)

        @pl.when(b2 == 0)
        def _():
            pltpu.make_async_copy(
                xs_hbm.at[src_v.at[b4]], rows_v.at[b3], ga).wait()

        @pl.when(b2 == 1)
        def _():
            pltpu.make_async_copy(
                xs_hbm.at[src_v.at[b4]], rows_v.at[b3], gb).wait()

        @pl.when(i > 0)
        def _():
            pltpu.make_async_copy(
                rows_v.at[b3p2], agg_sp.at[dst_v.at[b3p2]], ssem).wait()

        @pl.when(i == 0)
        def _():
            pltpu.make_async_copy(
                src_hbm.at[pl.ds(offp2, CH)], src_v.at[b4p2], s1sem).wait()

        pltpu.make_async_copy(
            src_hbm.at[pl.ds(offp2, CH)], src_v.at[b4p2], s1sem).wait()

        @pl.when(b2 == 0)
        def _():
            pltpu.make_async_copy(
                xs_hbm.at[src_v.at[b4p2]], rows_v.at[b3p2], ga).start()

        @pl.when(b2 == 1)
        def _():
            pltpu.make_async_copy(
                xs_hbm.at[src_v.at[b4p2]], rows_v.at[b3p2], gb).start()

        pltpu.make_async_copy(
            src_hbm.at[pl.ds(offp4, CH)], src_v.at[b4], s1sem).start()
        pltpu.make_async_copy(
            dst_hbm.at[pl.ds(offp2, CH)], dst_v.at[b3p2], s2sem).wait()
        pltpu.make_async_copy(
            rows_v.at[b3], agg_sp.at[dst_v.at[b3]], ssem).start(add=True)
        pltpu.make_async_copy(
            dst_hbm.at[pl.ds(offp2, CH)], dst_v.at[b3p2], s2sem).start()
        return carry

    lax.fori_loop(0, NCHUNK, body, 0)
    pltpu.make_async_copy(
        xs_hbm.at[src_v.at[NCHUNK % 4]], rows_v.at[NCHUNK % 3], ga).wait()
    pltpu.make_async_copy(
        xs_hbm.at[src_v.at[(NCHUNK + 1) % 4]],
        rows_v.at[(NCHUNK + 1) % 3], gb).wait()
    pltpu.make_async_copy(
        rows_v.at[(NCHUNK - 1) % 3],
        agg_sp.at[dst_v.at[(NCHUNK - 1) % 3]], ssem).wait()
    pltpu.make_async_copy(
        src_hbm.at[pl.ds(base, CH)], src_v.at[0], s1sem).wait()
    pltpu.make_async_copy(
        dst_hbm.at[pl.ds(base, CH)], dst_v.at[0], s2sem).wait()
    plsc.subcore_barrier()

    droff = pl.multiple_of(s * RPT, 8)
    nfull = RPT // CH
    sizes = [CH] * nfull + [RPT % CH]

    def _stage(k):
        return rows_v.at[k % 2, pl.ds(0, sizes[k])]

    def _wr(k):
        off = pl.multiple_of(droff + k * CH, 8)
        return pltpu.make_async_copy(
            _stage(k), out_hbm.at[c, pl.ds(off, sizes[k])],
            ga if k % 2 == 0 else gb)

    for k in range(nfull + 1):
        if k >= 2:
            _wr(k - 2).wait()
        off = pl.multiple_of(droff + k * CH, 8)
        pltpu.sync_copy(agg_sp.at[pl.ds(off, sizes[k])], _stage(k))
        _wr(k).start()
    _wr(nfull - 1).wait()
    _wr(nfull).wait()


def _prep_body(deg_ref, x_ref, xs_ref, ns_ref, nd_ref):
    d = deg_ref[...]
    deg_o = d[0, 0] + d[1, 0]
    deg_i = d[0, 1] + d[1, 1]
    ns = lax.rsqrt(jnp.maximum(deg_o, 1.0))
    nd = lax.rsqrt(jnp.maximum(deg_i, 1.0))
    xs_ref[...] = x_ref[...] * ns
    ns_ref[...] = ns
    nd_ref[...] = nd


_prep_call = pl.pallas_call(
    _prep_body,
    grid=(GRID,),
    in_specs=[
        pl.BlockSpec((NC, 2, BN, 1), lambda i: (0, 0, i, 0)),
        pl.BlockSpec((BN, D), lambda i: (i, 0)),
    ],
    out_specs=[
        pl.BlockSpec((BN, D), lambda i: (i, 0)),
        pl.BlockSpec((BN, 1), lambda i: (i, 0)),
        pl.BlockSpec((BN, 1), lambda i: (i, 0)),
    ],
    out_shape=[
        jax.ShapeDtypeStruct((N, D), jnp.float32),
        jax.ShapeDtypeStruct((N, 1), jnp.float32),
        jax.ShapeDtypeStruct((N, 1), jnp.float32),
    ],
)


def _mid_body(p_ref, nd_ref, ns_ref, w_ref, b_ref, o_ref):
    agg = (p_ref[0] + p_ref[1]) * nd_ref[...]
    z = jnp.dot(agg, w_ref[...], preferred_element_type=jnp.float32)
    z = jnp.maximum(z + b_ref[...], 0.0)
    o_ref[...] = z * ns_ref[...]


_mid_call = pl.pallas_call(
    _mid_body,
    grid=(GRID,),
    in_specs=[
        pl.BlockSpec((NC, BN, D), lambda i: (0, i, 0)),
        pl.BlockSpec((BN, 1), lambda i: (i, 0)),
        pl.BlockSpec((BN, 1), lambda i: (i, 0)),
        pl.BlockSpec((D, D), lambda i: (0, 0)),
        pl.BlockSpec((1, D), lambda i: (0, 0)),
    ],
    out_specs=pl.BlockSpec((BN, D), lambda i: (i, 0)),
    out_shape=jax.ShapeDtypeStruct((N, D), jnp.float32),
)


def _fin_body(p_ref, nd_ref, w_ref, b_ref, o_ref):
    agg = (p_ref[0] + p_ref[1]) * nd_ref[...]
    z = jnp.dot(agg, w_ref[...], preferred_element_type=jnp.float32)
    o_ref[...] = jnp.maximum(z + b_ref[...], 0.0)


_fin_call = pl.pallas_call(
    _fin_body,
    grid=(GRID,),
    in_specs=[
        pl.BlockSpec((NC, BN, D), lambda i: (0, i, 0)),
        pl.BlockSpec((BN, 1), lambda i: (i, 0)),
        pl.BlockSpec((D, D), lambda i: (0, 0)),
        pl.BlockSpec((1, D), lambda i: (0, 0)),
    ],
    out_specs=pl.BlockSpec((BN, D), lambda i: (i, 0)),
    out_shape=jax.ShapeDtypeStruct((N, D), jnp.float32),
)


def kernel(x, edge_index, W1, b1, W2, b2):
    src = edge_index[0]
    dst = edge_index[1]
    npad = EPAD - E
    iota = jnp.arange(npad, dtype=jnp.int32)
    src_s = jnp.concatenate([src, iota % 128])
    dst_s = jnp.concatenate([dst, N + iota % (NPAD - N)])
    diota = jnp.arange(DEPAD - E, dtype=jnp.int32)
    src_d = jnp.concatenate([src, 2 * N + diota % 2048])
    dst_d = jnp.concatenate([dst + N, 2 * N + diota % 2048])

    degs = _deg_kernel(src_d, dst_d)
    xs, ns, nd = _prep_call(degs.reshape(NC, 2, N, 1), x)

    p1 = _seg_kernel(xs, src_s, dst_s)
    zs = _mid_call(p1, nd, ns, W1, b1.reshape(1, D))

    p2 = _seg_kernel(zs, src_s, dst_s)
    out = _fin_call(p2, nd, W2, b2.reshape(1, D))
    return out


# speedup vs baseline: 3.7390x; 1.0069x over previous
import functools

import jax
import jax.numpy as jnp
from jax import lax
from jax.experimental import pallas as pl
from jax.experimental.pallas import tpu as pltpu
from jax.experimental.pallas import tpu_sc as plsc

N = 10000
E = 320000
D = 128

NC = 2
NS = 16
NW = NC * NS
CH = 128
EPW = 10240
NCHUNK = EPW // CH
EPAD = NW * EPW + 4 * CH
NPAD = 10112
RPT = NPAD // NS
DCH = 128
DEPW = 10240
DNCHUNK = DEPW // DCH
DEPAD = NW * DEPW + 4 * DCH

BN = 5000
GRID = N // BN

_mesh = plsc.VectorSubcoreMesh(core_axis_name="c", subcore_axis_name="s")


@functools.partial(
    pl.kernel,
    mesh=_mesh,
    out_type=jax.ShapeDtypeStruct((NC * 2 * N,), jnp.float32),
    scratch_types=[
        pltpu.VMEM_SHARED((2 * N + 2048,), jnp.float32),
        pltpu.VMEM((3, DCH), jnp.int32),
        pltpu.VMEM((3, DCH), jnp.int32),
        pltpu.VMEM((DCH,), jnp.float32),
        pltpu.VMEM((2000,), jnp.float32),
        pltpu.SemaphoreType.DMA,
        pltpu.SemaphoreType.DMA,
        pltpu.SemaphoreType.DMA,
    ],
)
def _deg_kernel(src_hbm, dstoff_hbm, out_hbm,
                deg_sp, src_v, dst_v, ones_v, stage_v, s1sem, s2sem, ssem):
    c = lax.axis_index("c")
    s = lax.axis_index("s")
    wid = c * NS + s
    base = wid * DEPW

    pltpu.sync_copy(src_hbm.at[pl.ds(base, DCH)], src_v.at[0])
    pltpu.sync_copy(dstoff_hbm.at[pl.ds(base, DCH)], dst_v.at[0])
    off1 = pl.multiple_of(base + DCH, 8)
    pltpu.make_async_copy(src_hbm.at[pl.ds(off1, DCH)], src_v.at[1], s1sem).start()
    pltpu.make_async_copy(dstoff_hbm.at[pl.ds(off1, DCH)], dst_v.at[1], s2sem).start()

    @pl.when(s < 10)
    def _():
        def zb(i, carry):
            stage_v[pl.ds(i * 16, 16)] = jnp.zeros((16,), jnp.float32)
            return carry
        lax.fori_loop(0, 2000 // 16, zb, 0)
        off = pl.multiple_of(s * 2000, 8)
        pltpu.sync_copy(stage_v, deg_sp.at[pl.ds(off, 2000)])

    for j in range(DCH // 16):
        ones_v[pl.ds(j * 16, 16)] = jnp.full((16,), 1.0, jnp.float32)

    plsc.subcore_barrier()

    def body(i, carry):
        b3 = lax.rem(i, 3)
        b3n = lax.rem(i + 1, 3)
        b3p = lax.rem(i + 2, 3)
        offp = pl.multiple_of(base + (i + 2) * DCH, 8)

        @pl.when(i > 0)
        def _():
            pltpu.make_async_copy(
                ones_v, deg_sp.at[src_v.at[b3p]], ssem).wait()
            pltpu.make_async_copy(
                ones_v, deg_sp.at[dst_v.at[b3p]], ssem).wait()

        pltpu.make_async_copy(
            src_hbm.at[pl.ds(offp, DCH)], src_v.at[b3], s1sem).wait()
        pltpu.make_async_copy(
            dstoff_hbm.at[pl.ds(offp, DCH)], dst_v.at[b3], s2sem).wait()
        pltpu.make_async_copy(
            ones_v, deg_sp.at[src_v.at[b3]], ssem).start(add=True)
        pltpu.make_async_copy(
            ones_v, deg_sp.at[dst_v.at[b3]], ssem).start(add=True)
        pltpu.make_async_copy(
            src_hbm.at[pl.ds(offp, DCH)], src_v.at[b3p], s1sem).start()
        pltpu.make_async_copy(
            dstoff_hbm.at[pl.ds(offp, DCH)], dst_v.at[b3p], s2sem).start()
        return carry

    lax.fori_loop(0, DNCHUNK, body, 0)
    pltpu.make_async_copy(
        ones_v, deg_sp.at[src_v.at[(DNCHUNK - 1) % 3]], ssem).wait()
    pltpu.make_async_copy(
        ones_v, deg_sp.at[dst_v.at[(DNCHUNK - 1) % 3]], ssem).wait()
    pltpu.make_async_copy(
        src_hbm.at[pl.ds(base, DCH)], src_v.at[0], s1sem).wait()
    pltpu.make_async_copy(
        dstoff_hbm.at[pl.ds(base, DCH)], dst_v.at[0], s2sem).wait()
    plsc.subcore_barrier()

    @pl.when(s < 10)
    def _():
        off = pl.multiple_of(s * 2000, 8)
        pltpu.sync_copy(deg_sp.at[pl.ds(off, 2000)], stage_v)
        pltpu.sync_copy(stage_v, out_hbm.at[pl.ds(c * 2 * N + off, 2000)])


@functools.partial(
    pl.kernel,
    mesh=_mesh,
    out_type=jax.ShapeDtypeStruct((NC, NPAD, D), jnp.float32),
    scratch_types=[
        pltpu.VMEM_SHARED((NPAD, D), jnp.float32),
        pltpu.VMEM((4, CH), jnp.int32),
        pltpu.VMEM((3, CH), jnp.int32),
        pltpu.VMEM((3, CH, D), jnp.float32),
        pltpu.SemaphoreType.DMA,
        pltpu.SemaphoreType.DMA,
        pltpu.SemaphoreType.DMA,
        pltpu.SemaphoreType.DMA,
        pltpu.SemaphoreType.DMA,
    ],
)
def _seg_kernel(xs_hbm, src_hbm, dst_hbm, out_hbm,
                agg_sp, src_v, dst_v, rows_v, ga, gb, s1sem, s2sem, ssem):
    c = lax.axis_index("c")
    s = lax.axis_index("s")
    wid = c * NS + s
    base = wid * EPW

    pltpu.sync_copy(src_hbm.at[pl.ds(base, CH)], src_v.at[0])
    pltpu.sync_copy(dst_hbm.at[pl.ds(base, CH)], dst_v.at[0])
    pltpu.make_async_copy(xs_hbm.at[src_v.at[0]], rows_v.at[0], ga).start()

    def _off(k):
        return pl.multiple_of(base + k * CH, 8)

    pltpu.make_async_copy(src_hbm.at[pl.ds(_off(1), CH)], src_v.at[1], s1sem).start()
    pltpu.make_async_copy(dst_hbm.at[pl.ds(_off(1), CH)], dst_v.at[1], s2sem).start()
    pltpu.make_async_copy(src_hbm.at[pl.ds(_off(1), CH)], src_v.at[1], s1sem).wait()
    pltpu.make_async_copy(xs_hbm.at[src_v.at[1]], rows_v.at[1], gb).start()
    pltpu.make_async_copy(src_hbm.at[pl.ds(_off(2), CH)], src_v.at[2], s1sem).start()
    pltpu.make_async_copy(src_hbm.at[pl.ds(_off(3), CH)], src_v.at[3], s1sem).start()

    def zb(i, carry):
        for j in range(D // 16):
            rows_v[2, i, pl.ds(j * 16, 16)] = jnp.zeros((16,), jnp.float32)
        return carry
    lax.fori_loop(0, CH, zb, 0)
    roff = pl.multiple_of(s * RPT, 8)
    for k in range(RPT // CH):
        pltpu.make_async_copy(
            rows_v.at[2], agg_sp.at[pl.ds(roff + k * CH, CH)], ssem).start()
    pltpu.make_async_copy(
        rows_v.at[2, pl.ds(0, RPT % CH)],
        agg_sp.at[pl.ds(roff + (RPT // CH) * CH, RPT % CH)], ssem).start()
    for k in range(RPT // CH):
        pltpu.make_async_copy(
            rows_v.at[2], agg_sp.at[pl.ds(roff + k * CH, CH)], ssem).wait()
    pltpu.make_async_copy(
        rows_v.at[2, pl.ds(0, RPT % CH)],
        agg_sp.at[pl.ds(roff + (RPT // CH) * CH, RPT % CH)], ssem).wait()

    plsc.subcore_barrier()

    def body(i, carry):
        b2 = lax.rem(i, 2)
        b3 = lax.rem(i, 3)
        b3p2 = lax.rem(i + 2, 3)
        b4 = lax.rem(i, 4)
        b4p2 = lax.rem(i + 2, 4)
        offp2 = pl.multiple_of(base + (i + 2) * CH, 8)
        offp4 = pl.multiple_of(base + (i + 4) * CH, 8)

        @pl.when(b2 == 0)
        def _():
            pltpu.make_async_copy(
                xs_hbm.at[src_v.at[b4]], rows_v.at[b3], ga).wait()

        @pl.when(b2 == 1)
        def _():
            pltpu.make_async_copy(
                xs_hbm.at[src_v.at[b4]], rows_v.at[b3], gb).wait()

        @pl.when(i > 0)
        def _():
            pltpu.make_async_copy(
                rows_v.at[b3p2], agg_sp.at[dst_v.at[b3p2]], ssem).wait()

        @pl.when(i == 0)
        def _():
            pltpu.make_async_copy(
                src_hbm.at[pl.ds(offp2, CH)], src_v.at[b4p2], s1sem).wait()

        pltpu.make_async_copy(
            src_hbm.at[pl.ds(offp2, CH)], src_v.at[b4p2], s1sem).wait()

        @pl.when(b2 == 0)
        def _():
            pltpu.make_async_copy(
                xs_hbm.at[src_v.at[b4p2]], rows_v.at[b3p2], ga).start()

        @pl.when(b2 == 1)
        def _():
            pltpu.make_async_copy(
                xs_hbm.at[src_v.at[b4p2]], rows_v.at[b3p2], gb).start()

        pltpu.make_async_copy(
            src_hbm.at[pl.ds(offp4, CH)], src_v.at[b4], s1sem).start()
        pltpu.make_async_copy(
            dst_hbm.at[pl.ds(offp2, CH)], dst_v.at[b3p2], s2sem).wait()
        pltpu.make_async_copy(
            rows_v.at[b3], agg_sp.at[dst_v.at[b3]], ssem).start(add=True)
        pltpu.make_async_copy(
            dst_hbm.at[pl.ds(offp2, CH)], dst_v.at[b3p2], s2sem).start()
        return carry

    lax.fori_loop(0, NCHUNK, body, 0)
    pltpu.make_async_copy(
        xs_hbm.at[src_v.at[NCHUNK % 4]], rows_v.at[NCHUNK % 3], ga).wait()
    pltpu.make_async_copy(
        xs_hbm.at[src_v.at[(NCHUNK + 1) % 4]],
        rows_v.at[(NCHUNK + 1) % 3], gb).wait()
    pltpu.make_async_copy(
        rows_v.at[(NCHUNK - 1) % 3],
        agg_sp.at[dst_v.at[(NCHUNK - 1) % 3]], ssem).wait()
    pltpu.make_async_copy(
        src_hbm.at[pl.ds(base, CH)], src_v.at[0], s1sem).wait()
    pltpu.make_async_copy(
        dst_hbm.at[pl.ds(base, CH)], dst_v.at[0], s2sem).wait()
    plsc.subcore_barrier()

    droff = pl.multiple_of(s * RPT, 8)
    nfull = RPT // CH
    sizes = [CH] * nfull + [RPT % CH]

    def _stage(k):
        return rows_v.at[k % 2, pl.ds(0, sizes[k])]

    def _wr(k):
        off = pl.multiple_of(droff + k * CH, 8)
        return pltpu.make_async_copy(
            _stage(k), out_hbm.at[c, pl.ds(off, sizes[k])],
            ga if k % 2 == 0 else gb)

    for k in range(nfull + 1):
        if k >= 2:
            _wr(k - 2).wait()
        off = pl.multiple_of(droff + k * CH, 8)
        pltpu.sync_copy(agg_sp.at[pl.ds(off, sizes[k])], _stage(k))
        _wr(k).start()
    _wr(nfull - 1).wait()
    _wr(nfull).wait()


def _prep_body(deg_ref, x_ref, xs_ref, ns_ref, nd_ref):
    d = deg_ref[...]
    deg_o = d[0, 0] + d[1, 0]
    deg_i = d[0, 1] + d[1, 1]
    ns = lax.rsqrt(jnp.maximum(deg_o, 1.0))
    nd = lax.rsqrt(jnp.maximum(deg_i, 1.0))
    xs_ref[...] = x_ref[...] * ns
    ns_ref[...] = ns
    nd_ref[...] = nd


_prep_call = pl.pallas_call(
    _prep_body,
    grid=(GRID,),
    in_specs=[
        pl.BlockSpec((NC, 2, BN, 1), lambda i: (0, 0, i, 0)),
        pl.BlockSpec((BN, D), lambda i: (i, 0)),
    ],
    out_specs=[
        pl.BlockSpec((BN, D), lambda i: (i, 0)),
        pl.BlockSpec((BN, 1), lambda i: (i, 0)),
        pl.BlockSpec((BN, 1), lambda i: (i, 0)),
    ],
    out_shape=[
        jax.ShapeDtypeStruct((N, D), jnp.float32),
        jax.ShapeDtypeStruct((N, 1), jnp.float32),
        jax.ShapeDtypeStruct((N, 1), jnp.float32),
    ],
)


def _mid_body(p_ref, nd_ref, ns_ref, w_ref, b_ref, o_ref):
    agg = (p_ref[0] + p_ref[1]) * nd_ref[...]
    z = jnp.dot(agg, w_ref[...], preferred_element_type=jnp.float32)
    z = jnp.maximum(z + b_ref[...], 0.0)
    o_ref[...] = z * ns_ref[...]


_mid_call = pl.pallas_call(
    _mid_body,
    grid=(GRID,),
    in_specs=[
        pl.BlockSpec((NC, BN, D), lambda i: (0, i, 0)),
        pl.BlockSpec((BN, 1), lambda i: (i, 0)),
        pl.BlockSpec((BN, 1), lambda i: (i, 0)),
        pl.BlockSpec((D, D), lambda i: (0, 0)),
        pl.BlockSpec((1, D), lambda i: (0, 0)),
    ],
    out_specs=pl.BlockSpec((BN, D), lambda i: (i, 0)),
    out_shape=jax.ShapeDtypeStruct((N, D), jnp.float32),
)


def _fin_body(p_ref, nd_ref, w_ref, b_ref, o_ref):
    agg = (p_ref[0] + p_ref[1]) * nd_ref[...]
    z = jnp.dot(agg, w_ref[...], preferred_element_type=jnp.float32)
    o_ref[...] = jnp.maximum(z + b_ref[...], 0.0)


_fin_call = pl.pallas_call(
    _fin_body,
    grid=(GRID,),
    in_specs=[
        pl.BlockSpec((NC, BN, D), lambda i: (0, i, 0)),
        pl.BlockSpec((BN, 1), lambda i: (i, 0)),
        pl.BlockSpec((D, D), lambda i: (0, 0)),
        pl.BlockSpec((1, D), lambda i: (0, 0)),
    ],
    out_specs=pl.BlockSpec((BN, D), lambda i: (i, 0)),
    out_shape=jax.ShapeDtypeStruct((N, D), jnp.float32),
)


def kernel(x, edge_index, W1, b1, W2, b2):
    src = edge_index[0]
    dst = edge_index[1]
    npad = EPAD - E
    iota = jnp.arange(npad, dtype=jnp.int32)
    src_s = jnp.concatenate([src, iota % 128])
    dst_s = jnp.concatenate([dst, N + iota % (NPAD - N)])
    diota = jnp.arange(DEPAD - E, dtype=jnp.int32)
    src_d = jnp.concatenate([src, 2 * N + diota % 2048])
    dst_d = jnp.concatenate([dst + N, 2 * N + diota % 2048])

    degs = _deg_kernel(src_d, dst_d)
    xs, ns, nd = _prep_call(degs.reshape(NC, 2, N, 1), x)

    p1 = _seg_kernel(xs, src_s, dst_s)
    zs = _mid_call(p1, nd, ns, W1, b1.reshape(1, D))

    p2 = _seg_kernel(zs, src_s, dst_s)
    out = _fin_call(p2, nd, W2, b2.reshape(1, D))
    return out
